# Initial kernel scaffold; baseline (speedup 1.0000x reference)
import jax
import jax.numpy as jnp
from jax.experimental import pallas as pl


def kernel(x, edge_index, W1, a_src1, a_dst1, b1, g1, be1, W2, a_src2, a_dst2, b2, g2, be2):
    raise NotImplementedError("write your pallas kernel here")


# speedup vs baseline: 1.1445x; 1.1445x over previous
import functools

import jax
import jax.numpy as jnp
from jax.experimental import pallas as pl

HEADS = 2


def _bn_relu_kernel(x_ref, g_ref, be_ref, o_ref):
    x = x_ref[...]
    mu = jnp.mean(x, axis=0, keepdims=True)
    var = jnp.mean((x - mu) ** 2, axis=0, keepdims=True)
    y = (x - mu) * jax.lax.rsqrt(var + 1e-5) * g_ref[...] + be_ref[...]
    o_ref[...] = jnp.maximum(y, 0.0)


def _bn_relu(x, g, be):
    n, c = x.shape
    blk = 128
    return pl.pallas_call(
        _bn_relu_kernel,
        grid=(c // blk,),
        in_specs=[
            pl.BlockSpec((n, blk), lambda j: (0, j)),
            pl.BlockSpec((1, blk), lambda j: (0, j)),
            pl.BlockSpec((1, blk), lambda j: (0, j)),
        ],
        out_specs=pl.BlockSpec((n, blk), lambda j: (0, j)),
        out_shape=jax.ShapeDtypeStruct((n, c), x.dtype),
    )(x, g.reshape(1, c), be.reshape(1, c))


def _gat_layer(x, src, dst, W, a_src, a_dst, b):
    n = x.shape[0]
    c = a_src.shape[1]
    h = (x @ W).reshape(n, HEADS, c)
    asrc = jnp.sum(h * a_src[None], axis=-1)
    adst = jnp.sum(h * a_dst[None], axis=-1)
    e = jax.nn.leaky_relu(asrc[src] + adst[dst], 0.2)
    w = jnp.exp(e)
    wself = jnp.exp(jax.nn.leaky_relu(asrc + adst, 0.2))
    num = jax.ops.segment_sum(w[:, :, None] * h[src], dst, num_segments=n)
    num = num + wself[:, :, None] * h
    den = jax.ops.segment_sum(w, dst, num_segments=n) + wself
    out = num / (den[:, :, None] + 1e-16)
    return out.reshape(n, HEADS * c) + b


def kernel(x, edge_index, W1, a_src1, a_dst1, b1, g1, be1,
           W2, a_src2, a_dst2, b2, g2, be2):
    src = edge_index[0].astype(jnp.int32)
    dst = edge_index[1].astype(jnp.int32)
    h = _gat_layer(x, src, dst, W1, a_src1, a_dst1, b1)
    h = _bn_relu(h, g1, be1)
    h = _gat_layer(h, src, dst, W2, a_src2, a_dst2, b2)
    h = _bn_relu(h, g2, be2)
    return h


# speedup vs baseline: 3.0831x; 2.6938x over previous
import functools

import jax
import jax.numpy as jnp
from jax import lax
from jax.experimental import pallas as pl
from jax.experimental.pallas import tpu as pltpu
from jax.experimental.pallas import tpu_sc as plsc

HEADS = 2
N = 10000
E = 160000
NPAD = 10240
ROWBLK = 1280
EPT = E // 16
ECHUNK = 2000
NCHUNK = EPT // ECHUNK
CCAP = 2048
SUB = 32


def _stage_a_body(C, x_ref, w_ref, as_ref, ad_ref, h_ref, at_ref, n0_ref,
                  d0_ref):
    x = x_ref[...]
    h = jnp.dot(x, w_ref[...], preferred_element_type=jnp.float32)
    h0 = h[:, :C]
    h1 = h[:, C:]
    a_s = as_ref[...]
    a_d = ad_ref[...]
    asrc0 = jnp.sum(h0 * a_s[0:1, :], axis=1)
    asrc1 = jnp.sum(h1 * a_s[1:2, :], axis=1)
    adst0 = jnp.sum(h0 * a_d[0:1, :], axis=1)
    adst1 = jnp.sum(h1 * a_d[1:2, :], axis=1)
    s0 = asrc0 + adst0
    s1 = asrc1 + adst1
    ws0 = jnp.exp(jnp.where(s0 >= 0, s0, 0.2 * s0))
    ws1 = jnp.exp(jnp.where(s1 >= 0, s1, 0.2 * s1))
    h_ref[...] = h
    z = jnp.zeros((1, x.shape[0]), jnp.float32)
    at_ref[...] = jnp.concatenate(
        [asrc0[None], asrc1[None], adst0[None], adst1[None], z, z, z, z], 0)
    n0_ref[...] = jnp.concatenate([h0 * ws0[:, None], h1 * ws1[:, None]], 1)
    d0_ref[...] = jnp.concatenate(
        [ws0[:, None], ws1[:, None],
         jnp.zeros((x.shape[0], 14), jnp.float32)], 1)


def _stage_a(x_p, W, a_src, a_dst):
    cin = x_p.shape[1]
    C = a_src.shape[1]
    HC = HEADS * C
    nblk = NPAD // ROWBLK
    return pl.pallas_call(
        functools.partial(_stage_a_body, C),
        grid=(nblk,),
        in_specs=[
            pl.BlockSpec((ROWBLK, cin), lambda j: (j, 0)),
            pl.BlockSpec((cin, HC), lambda j: (0, 0)),
            pl.BlockSpec((HEADS, C), lambda j: (0, 0)),
            pl.BlockSpec((HEADS, C), lambda j: (0, 0)),
        ],
        out_specs=[
            pl.BlockSpec((ROWBLK, HC), lambda j: (j, 0)),
            pl.BlockSpec((8, ROWBLK), lambda j: (0, j)),
            pl.BlockSpec((ROWBLK, HC), lambda j: (j, 0)),
            pl.BlockSpec((ROWBLK, 16), lambda j: (j, 0)),
        ],
        out_shape=[
            jax.ShapeDtypeStruct((NPAD, HC), jnp.float32),
            jax.ShapeDtypeStruct((8, NPAD), jnp.float32),
            jax.ShapeDtypeStruct((NPAD, HC), jnp.float32),
            jax.ShapeDtypeStruct((NPAD, 16), jnp.float32),
        ],
    )(x_p, W, a_src, a_dst)


def _iota16():
    return lax.iota(jnp.int32, 16)


def _sc_w_body(srcr, dstr, alphat, w0o, w1o,
               as0_v, as1_v, ad0_v, ad1_v, src_v, dst_v, w0_v, w1_v):
    cid = lax.axis_index("c")
    sid = lax.axis_index("s")
    pltpu.sync_copy(alphat.at[0], as0_v)
    pltpu.sync_copy(alphat.at[1], as1_v)
    pltpu.sync_copy(alphat.at[2], ad0_v)
    pltpu.sync_copy(alphat.at[3], ad1_v)
    for c in range(NCHUNK):
        @pl.when((c % 2) == cid)
        def _():
            ebase = pl.multiple_of(sid * EPT + c * ECHUNK, 8)
            pltpu.sync_copy(srcr.at[pl.ds(ebase, ECHUNK)], src_v)
            pltpu.sync_copy(dstr.at[pl.ds(ebase, ECHUNK)], dst_v)

            def vb(i, carry):
                o = pl.multiple_of(i * 16, 16)
                s16 = src_v[pl.ds(o, 16)]
                d16 = dst_v[pl.ds(o, 16)]
                a0 = plsc.load_gather(as0_v, [s16])
                a1 = plsc.load_gather(as1_v, [s16])
                b0 = plsc.load_gather(ad0_v, [d16])
                b1 = plsc.load_gather(ad1_v, [d16])
                e0 = a0 + b0
                e1 = a1 + b1
                w0_v[pl.ds(o, 16)] = jnp.exp(
                    jnp.where(e0 >= 0, e0, 0.2 * e0))
                w1_v[pl.ds(o, 16)] = jnp.exp(
                    jnp.where(e1 >= 0, e1, 0.2 * e1))
                return carry

            lax.fori_loop(0, ECHUNK // 16, vb, 0)
            pltpu.sync_copy(w0_v, w0o.at[pl.ds(ebase, ECHUNK)])
            pltpu.sync_copy(w1_v, w1o.at[pl.ds(ebase, ECHUNK)])


def _sc_weights(src, dst, alphat):
    mesh = plsc.VectorSubcoreMesh(core_axis_name="c", subcore_axis_name="s")
    f = pl.kernel(
        _sc_w_body,
        mesh=mesh,
        compiler_params=pltpu.CompilerParams(needs_layout_passes=False),
        out_type=[
            jax.ShapeDtypeStruct((E,), jnp.float32),
            jax.ShapeDtypeStruct((E,), jnp.float32),
        ],
        scratch_types=[
            pltpu.VMEM((NPAD,), jnp.float32),
            pltpu.VMEM((NPAD,), jnp.float32),
            pltpu.VMEM((NPAD,), jnp.float32),
            pltpu.VMEM((NPAD,), jnp.float32),
            pltpu.VMEM((ECHUNK,), jnp.int32),
            pltpu.VMEM((ECHUNK,), jnp.int32),
            pltpu.VMEM((ECHUNK,), jnp.float32),
            pltpu.VMEM((ECHUNK,), jnp.float32),
        ],
    )
    return f(src, dst, alphat)


def _sc_agg_body(C, R, n_passes,
                 h, srcr, dstr, w0r, w1r, num0, den0, numf, denf,
                 src_v, dst_v, w0c, w1c,
                 csrc, cdl, cw0, cw1, rows_v, gidx,
                 accf, accd):
    HC = HEADS * C
    cid = lax.axis_index("c")
    sid = lax.axis_index("s")
    wid = sid * 2 + cid
    zc16 = jnp.zeros((16,), jnp.int32)
    zf16 = jnp.zeros((16,), jnp.float32)
    iota = _iota16()

    for p in range(n_passes):
        lo = (p * 32 + wid) * R
        pltpu.sync_copy(num0.at[pl.ds(lo, R)], accf)
        pltpu.sync_copy(den0.at[pl.ds(pl.multiple_of(lo * 16, 16), R * 16)],
                        accd)

        def echunk(chunk, carry0):
            ebase = pl.multiple_of(chunk * ECHUNK, 8)
            pltpu.sync_copy(srcr.at[pl.ds(ebase, ECHUNK)], src_v)
            pltpu.sync_copy(dstr.at[pl.ds(ebase, ECHUNK)], dst_v)
            pltpu.sync_copy(w0r.at[pl.ds(ebase, ECHUNK)], w0c)
            pltpu.sync_copy(w1r.at[pl.ds(ebase, ECHUNK)], w1c)

            def vbody(i, off):
                o = pl.multiple_of(i * 16, 16)
                s16 = src_v[pl.ds(o, 16)]
                d16 = dst_v[pl.ds(o, 16)]
                w0 = w0c[pl.ds(o, 16)]
                w1 = w1c[pl.ds(o, 16)]
                dl = d16 - lo
                m = (dl >= 0) & (dl < R)
                mi = m.astype(jnp.int32)
                pos = off + plsc.cumsum(mi) - mi
                plsc.store_scatter(csrc, [pos], s16, mask=m)
                plsc.store_scatter(cdl, [pos], dl, mask=m)
                plsc.store_scatter(cw0, [pos], w0, mask=m)
                plsc.store_scatter(cw1, [pos], w1, mask=m)
                return off + jnp.sum(mi)

            off = lax.fori_loop(0, ECHUNK // 16, vbody, jnp.int32(0))

            nsub = (off + (SUB - 1)) // SUB
            pad_end = nsub * SUB
            for k in range(SUB // 16):
                idx = off + k * 16 + iota
                pm = idx < pad_end
                plsc.store_scatter(csrc, [idx], zc16, mask=pm)
                plsc.store_scatter(cdl, [idx], zc16, mask=pm)
                plsc.store_scatter(cw0, [idx], zf16, mask=pm)
                plsc.store_scatter(cw1, [idx], zf16, mask=pm)

            def drain(j, carry):
                for k in range(SUB // 16):
                    ok = pl.multiple_of(j * SUB + k * 16, 16)
                    gidx[pl.ds(k * 16, 16)] = csrc[pl.ds(ok, 16)]
                pltpu.sync_copy(h.at[gidx], rows_v)

                def srow(r, c2):
                    i = j * SUB + r
                    si = jnp.full((16,), i, jnp.int32)
                    rs = jnp.full((16,), r, jnp.int32)
                    dlv = plsc.load_gather(cdl, [si])
                    f0 = plsc.load_gather(cw0, [si])
                    f1 = plsc.load_gather(cw1, [si])
                    wrow = jnp.where(iota == 0, f0,
                                     jnp.where(iota == 1, f1, zf16))
                    plsc.addupdate_scatter(accd, [dlv * 16 + iota], wrow)
                    for q in range(HC // 16):
                        cq = q * 16 + iota
                        f = f0 if q < (C // 16) else f1
                        v = plsc.load_gather(rows_v, [rs, cq]) * f
                        plsc.addupdate_scatter(accf, [dlv, cq], v)
                    return c2

                lax.fori_loop(0, SUB, srow, 0)
                return carry

            lax.fori_loop(0, nsub, drain, 0)
            return carry0

        lax.fori_loop(0, E // ECHUNK, echunk, 0)

        pltpu.sync_copy(accf, numf.at[pl.ds(lo, R)])
        pltpu.sync_copy(accd,
                        denf.at[pl.ds(pl.multiple_of(lo * 16, 16), R * 16)])


def _sc_agg(C, R, n_passes, h, src, dst, w0, w1, num0, den0):
    HC = HEADS * C
    mesh = plsc.VectorSubcoreMesh(core_axis_name="c", subcore_axis_name="s")
    f = pl.kernel(
        functools.partial(_sc_agg_body, C, R, n_passes),
        mesh=mesh,
        compiler_params=pltpu.CompilerParams(needs_layout_passes=False),
        out_type=[
            jax.ShapeDtypeStruct((NPAD, HC), jnp.float32),
            jax.ShapeDtypeStruct((NPAD * 16,), jnp.float32),
        ],
        scratch_types=[
            pltpu.VMEM((ECHUNK,), jnp.int32),
            pltpu.VMEM((ECHUNK,), jnp.int32),
            pltpu.VMEM((ECHUNK,), jnp.float32),
            pltpu.VMEM((ECHUNK,), jnp.float32),
            pltpu.VMEM((CCAP,), jnp.int32),
            pltpu.VMEM((CCAP,), jnp.int32),
            pltpu.VMEM((CCAP,), jnp.float32),
            pltpu.VMEM((CCAP,), jnp.float32),
            pltpu.VMEM((SUB, HC), jnp.float32),
            pltpu.VMEM((SUB,), jnp.int32),
            pltpu.VMEM((R, HC), jnp.float32),
            pltpu.VMEM((R * 16,), jnp.float32),
        ],
    )
    return f(h, src, dst, w0, w1, num0, den0)


def _stage_c_body(nbh, num_ref, den_ref, b_ref, g_ref, be_ref, o_ref):
    j = pl.program_id(0)
    num = num_ref[...]
    den = den_ref[...]
    d = jnp.where(j < nbh, den[:, 0:1], den[:, 1:2])
    pre = num / (d + 1e-16) + b_ref[...]
    mu = jnp.mean(pre, axis=0, keepdims=True)
    var = jnp.mean((pre - mu) ** 2, axis=0, keepdims=True)
    y = (pre - mu) * lax.rsqrt(var + 1e-5) * g_ref[...] + be_ref[...]
    o_ref[...] = jnp.maximum(y, 0.0)


def _stage_c(numf, denf, b, g, be):
    HC = b.shape[0]
    C = HC // HEADS
    nblk = HC // 128
    return pl.pallas_call(
        functools.partial(_stage_c_body, C // 128),
        grid=(nblk,),
        in_specs=[
            pl.BlockSpec((N, 128), lambda j: (0, j)),
            pl.BlockSpec((N, 16), lambda j: (0, 0)),
            pl.BlockSpec((1, 128), lambda j: (0, j)),
            pl.BlockSpec((1, 128), lambda j: (0, j)),
            pl.BlockSpec((1, 128), lambda j: (0, j)),
        ],
        out_specs=pl.BlockSpec((N, 128), lambda j: (0, j)),
        out_shape=jax.ShapeDtypeStruct((N, HC), jnp.float32),
    )(numf, denf, b.reshape(1, HC), g.reshape(1, HC), be.reshape(1, HC))


def kernel(x, edge_index, W1, a_src1, a_dst1, b1, g1, be1,
           W2, a_src2, a_dst2, b2, g2, be2):
    src = edge_index[0].astype(jnp.int32)
    dst = edge_index[1].astype(jnp.int32)

    def layer(x_in, W, a_src, a_dst, b, g, be, R, n_passes):
        x_p = jnp.pad(x_in, ((0, NPAD - N), (0, 0)))
        h, at, n0, d0 = _stage_a(x_p, W, a_src, a_dst)
        w0, w1 = _sc_weights(src, dst, at)
        numf, denf = _sc_agg(a_src.shape[1], R, n_passes,
                             h, src, dst, w0, w1, n0, d0.reshape(-1))
        return _stage_c(numf[:N], denf.reshape(NPAD, 16)[:N], b, g, be)

    t = layer(x, W1, a_src1, a_dst1, b1, g1, be1, 160, 2)
    return layer(t, W2, a_src2, a_dst2, b2, g2, be2, 64, 5)


# speedup vs baseline: 6.3998x; 2.0758x over previous
import functools

import jax
import jax.numpy as jnp
from jax import lax
from jax.experimental import pallas as pl
from jax.experimental.pallas import tpu as pltpu
from jax.experimental.pallas import tpu_sc as plsc

HEADS = 2
N = 10000
E = 160000
NPAD = 10240
ROWBLK = 1280
EPT = E // 16
ECHUNK = 2000
NCHUNK = EPT // ECHUNK
CCAP = 2048
SUB = 16


def _stage_a_body(C, x_ref, w_ref, as_ref, ad_ref, h_ref, at_ref, n0_ref,
                  d0_ref):
    x = x_ref[...]
    h = jnp.dot(x, w_ref[...], preferred_element_type=jnp.float32)
    h0 = h[:, :C]
    h1 = h[:, C:]
    a_s = as_ref[...]
    a_d = ad_ref[...]
    asrc0 = jnp.sum(h0 * a_s[0:1, :], axis=1)
    asrc1 = jnp.sum(h1 * a_s[1:2, :], axis=1)
    adst0 = jnp.sum(h0 * a_d[0:1, :], axis=1)
    adst1 = jnp.sum(h1 * a_d[1:2, :], axis=1)
    s0 = asrc0 + adst0
    s1 = asrc1 + adst1
    ws0 = jnp.exp(jnp.where(s0 >= 0, s0, 0.2 * s0))
    ws1 = jnp.exp(jnp.where(s1 >= 0, s1, 0.2 * s1))
    h_ref[...] = h
    z = jnp.zeros((1, x.shape[0]), jnp.float32)
    at_ref[...] = jnp.concatenate(
        [asrc0[None], asrc1[None], adst0[None], adst1[None], z, z, z, z], 0)
    n0_ref[...] = jnp.concatenate([h0 * ws0[:, None], h1 * ws1[:, None]], 1)
    d0_ref[...] = jnp.concatenate(
        [ws0[:, None], ws1[:, None],
         jnp.zeros((x.shape[0], 14), jnp.float32)], 1)


def _stage_a(x_p, W, a_src, a_dst):
    cin = x_p.shape[1]
    C = a_src.shape[1]
    HC = HEADS * C
    nblk = NPAD // ROWBLK
    return pl.pallas_call(
        functools.partial(_stage_a_body, C),
        grid=(nblk,),
        in_specs=[
            pl.BlockSpec((ROWBLK, cin), lambda j: (j, 0)),
            pl.BlockSpec((cin, HC), lambda j: (0, 0)),
            pl.BlockSpec((HEADS, C), lambda j: (0, 0)),
            pl.BlockSpec((HEADS, C), lambda j: (0, 0)),
        ],
        out_specs=[
            pl.BlockSpec((ROWBLK, HC), lambda j: (j, 0)),
            pl.BlockSpec((8, ROWBLK), lambda j: (0, j)),
            pl.BlockSpec((ROWBLK, HC), lambda j: (j, 0)),
            pl.BlockSpec((ROWBLK, 16), lambda j: (j, 0)),
        ],
        out_shape=[
            jax.ShapeDtypeStruct((NPAD, HC), jnp.float32),
            jax.ShapeDtypeStruct((8, NPAD), jnp.float32),
            jax.ShapeDtypeStruct((NPAD, HC), jnp.float32),
            jax.ShapeDtypeStruct((NPAD, 16), jnp.float32),
        ],
    )(x_p, W, a_src, a_dst)


def _iota16():
    return lax.iota(jnp.int32, 16)


def _sc_w_body(srcr, dstr, alphat, w0o, w1o,
               as0_v, as1_v, ad0_v, ad1_v, src_v, dst_v, w0_v, w1_v):
    cid = lax.axis_index("c")
    sid = lax.axis_index("s")
    pltpu.sync_copy(alphat.at[0], as0_v)
    pltpu.sync_copy(alphat.at[1], as1_v)
    pltpu.sync_copy(alphat.at[2], ad0_v)
    pltpu.sync_copy(alphat.at[3], ad1_v)
    for c in range(NCHUNK):
        @pl.when((c % 2) == cid)
        def _():
            ebase = pl.multiple_of(sid * EPT + c * ECHUNK, 8)
            pltpu.sync_copy(srcr.at[pl.ds(ebase, ECHUNK)], src_v)
            pltpu.sync_copy(dstr.at[pl.ds(ebase, ECHUNK)], dst_v)

            def vb(i, carry):
                o = pl.multiple_of(i * 16, 16)
                s16 = src_v[pl.ds(o, 16)]
                d16 = dst_v[pl.ds(o, 16)]
                a0 = plsc.load_gather(as0_v, [s16])
                a1 = plsc.load_gather(as1_v, [s16])
                b0 = plsc.load_gather(ad0_v, [d16])
                b1 = plsc.load_gather(ad1_v, [d16])
                e0 = a0 + b0
                e1 = a1 + b1
                w0_v[pl.ds(o, 16)] = jnp.exp(
                    jnp.where(e0 >= 0, e0, 0.2 * e0))
                w1_v[pl.ds(o, 16)] = jnp.exp(
                    jnp.where(e1 >= 0, e1, 0.2 * e1))
                return carry

            lax.fori_loop(0, ECHUNK // 16, vb, 0)
            pltpu.sync_copy(w0_v, w0o.at[pl.ds(ebase, ECHUNK)])
            pltpu.sync_copy(w1_v, w1o.at[pl.ds(ebase, ECHUNK)])


def _sc_weights(src, dst, alphat):
    mesh = plsc.VectorSubcoreMesh(core_axis_name="c", subcore_axis_name="s")
    f = pl.kernel(
        _sc_w_body,
        mesh=mesh,
        compiler_params=pltpu.CompilerParams(needs_layout_passes=False),
        out_type=[
            jax.ShapeDtypeStruct((E,), jnp.float32),
            jax.ShapeDtypeStruct((E,), jnp.float32),
        ],
        scratch_types=[
            pltpu.VMEM((NPAD,), jnp.float32),
            pltpu.VMEM((NPAD,), jnp.float32),
            pltpu.VMEM((NPAD,), jnp.float32),
            pltpu.VMEM((NPAD,), jnp.float32),
            pltpu.VMEM((ECHUNK,), jnp.int32),
            pltpu.VMEM((ECHUNK,), jnp.int32),
            pltpu.VMEM((ECHUNK,), jnp.float32),
            pltpu.VMEM((ECHUNK,), jnp.float32),
        ],
    )
    return f(src, dst, alphat)


def _sc_agg_body(C, R, n_passes,
                 h, srcr, dstr, w0r, w1r, num0, den0, numf, denf,
                 src_v, dst_v, w0c, w1c,
                 csrc, cdl, cw0, cw1, rows_v, gidx,
                 accf, accd):
    HC = HEADS * C
    cid = lax.axis_index("c")
    sid = lax.axis_index("s")
    wid = sid * 2 + cid
    zc16 = jnp.zeros((16,), jnp.int32)
    zf16 = jnp.zeros((16,), jnp.float32)
    iota = _iota16()

    for p in range(n_passes):
        lo = (p * 32 + wid) * R
        pltpu.sync_copy(num0.at[pl.ds(lo, R)], accf)
        pltpu.sync_copy(den0.at[pl.ds(pl.multiple_of(lo * 16, 16), R * 16)],
                        accd)

        def echunk(chunk, carry0):
            ebase = pl.multiple_of(chunk * ECHUNK, 8)
            pltpu.sync_copy(srcr.at[pl.ds(ebase, ECHUNK)], src_v)
            pltpu.sync_copy(dstr.at[pl.ds(ebase, ECHUNK)], dst_v)
            pltpu.sync_copy(w0r.at[pl.ds(ebase, ECHUNK)], w0c)
            pltpu.sync_copy(w1r.at[pl.ds(ebase, ECHUNK)], w1c)

            def vbody(i, off):
                o = pl.multiple_of(i * 16, 16)
                s16 = src_v[pl.ds(o, 16)]
                d16 = dst_v[pl.ds(o, 16)]
                w0 = w0c[pl.ds(o, 16)]
                w1 = w1c[pl.ds(o, 16)]
                dl = d16 - lo
                m = (dl >= 0) & (dl < R)
                mi = m.astype(jnp.int32)
                pos = off + plsc.cumsum(mi) - mi
                plsc.store_scatter(csrc, [pos], s16, mask=m)
                plsc.store_scatter(cdl, [pos], dl, mask=m)
                plsc.store_scatter(cw0, [pos], w0, mask=m)
                plsc.store_scatter(cw1, [pos], w1, mask=m)
                return off + jnp.sum(mi)

            off = lax.fori_loop(0, ECHUNK // 16, vbody, jnp.int32(0))

            nsub = (off + (SUB - 1)) // SUB
            pad_end = nsub * SUB
            for k in range(SUB // 16):
                idx = off + k * 16 + iota
                pm = idx < pad_end
                plsc.store_scatter(csrc, [idx], zc16, mask=pm)
                plsc.store_scatter(cdl, [idx], zc16, mask=pm)
                plsc.store_scatter(cw0, [idx], zf16, mask=pm)
                plsc.store_scatter(cw1, [idx], zf16, mask=pm)

            def drain(j, carry):
                for k in range(SUB // 16):
                    ok = pl.multiple_of(j * SUB + k * 16, 16)
                    gidx[pl.ds(k * 16, 16)] = csrc[pl.ds(ok, 16)]
                pltpu.sync_copy(h.at[gidx], rows_v)

                def srow(r, c2):
                    i = j * SUB + r
                    si = jnp.full((16,), i, jnp.int32)
                    rs = jnp.full((16,), r, jnp.int32)
                    dlv = plsc.load_gather(cdl, [si])
                    f0 = plsc.load_gather(cw0, [si])
                    f1 = plsc.load_gather(cw1, [si])
                    wrow = jnp.where(iota == 0, f0,
                                     jnp.where(iota == 1, f1, zf16))
                    plsc.addupdate_scatter(accd, [dlv * 16 + iota], wrow)
                    for q in range(HC // 16):
                        cq = q * 16 + iota
                        f = f0 if q < (C // 16) else f1
                        v = plsc.load_gather(rows_v, [rs, cq]) * f
                        plsc.addupdate_scatter(accf, [dlv, cq], v)
                    return c2

                lax.fori_loop(0, SUB, srow, 0)
                return carry

            lax.fori_loop(0, nsub, drain, 0)
            return carry0

        lax.fori_loop(0, E // ECHUNK, echunk, 0)

        pltpu.sync_copy(accf, numf.at[pl.ds(lo, R)])
        pltpu.sync_copy(accd,
                        denf.at[pl.ds(pl.multiple_of(lo * 16, 16), R * 16)])


def _sc_agg(C, R, n_passes, h, src, dst, w0, w1, num0, den0):
    HC = HEADS * C
    mesh = plsc.VectorSubcoreMesh(core_axis_name="c", subcore_axis_name="s")
    f = pl.kernel(
        functools.partial(_sc_agg_body, C, R, n_passes),
        mesh=mesh,
        compiler_params=pltpu.CompilerParams(needs_layout_passes=False),
        out_type=[
            jax.ShapeDtypeStruct((NPAD, HC), jnp.float32),
            jax.ShapeDtypeStruct((NPAD * 16,), jnp.float32),
        ],
        scratch_types=[
            pltpu.VMEM((ECHUNK,), jnp.int32),
            pltpu.VMEM((ECHUNK,), jnp.int32),
            pltpu.VMEM((ECHUNK,), jnp.float32),
            pltpu.VMEM((ECHUNK,), jnp.float32),
            pltpu.VMEM((CCAP,), jnp.int32),
            pltpu.VMEM((CCAP,), jnp.int32),
            pltpu.VMEM((CCAP,), jnp.float32),
            pltpu.VMEM((CCAP,), jnp.float32),
            pltpu.VMEM((SUB, HC), jnp.float32),
            pltpu.VMEM((SUB,), jnp.int32),
            pltpu.VMEM((R, HC), jnp.float32),
            pltpu.VMEM((R * 16,), jnp.float32),
        ],
    )
    return f(h, src, dst, w0, w1, num0, den0)


def _stage_c_body(nbh, num_ref, den_ref, b_ref, g_ref, be_ref, o_ref):
    j = pl.program_id(0)
    num = num_ref[...]
    den = den_ref[...]
    d = jnp.where(j < nbh, den[:, 0:1], den[:, 1:2])
    pre = num / (d + 1e-16) + b_ref[...]
    mu = jnp.mean(pre, axis=0, keepdims=True)
    var = jnp.mean((pre - mu) ** 2, axis=0, keepdims=True)
    y = (pre - mu) * lax.rsqrt(var + 1e-5) * g_ref[...] + be_ref[...]
    o_ref[...] = jnp.maximum(y, 0.0)


def _stage_c(numf, denf, b, g, be):
    HC = b.shape[0]
    C = HC // HEADS
    nblk = HC // 128
    return pl.pallas_call(
        functools.partial(_stage_c_body, C // 128),
        grid=(nblk,),
        in_specs=[
            pl.BlockSpec((N, 128), lambda j: (0, j)),
            pl.BlockSpec((N, 16), lambda j: (0, 0)),
            pl.BlockSpec((1, 128), lambda j: (0, j)),
            pl.BlockSpec((1, 128), lambda j: (0, j)),
            pl.BlockSpec((1, 128), lambda j: (0, j)),
        ],
        out_specs=pl.BlockSpec((N, 128), lambda j: (0, j)),
        out_shape=jax.ShapeDtypeStruct((N, HC), jnp.float32),
    )(numf, denf, b.reshape(1, HC), g.reshape(1, HC), be.reshape(1, HC))


def kernel(x, edge_index, W1, a_src1, a_dst1, b1, g1, be1,
           W2, a_src2, a_dst2, b2, g2, be2):
    src = edge_index[0].astype(jnp.int32)
    dst = edge_index[1].astype(jnp.int32)

    def layer(x_in, W, a_src, a_dst, b, g, be, R, n_passes):
        x_p = jnp.pad(x_in, ((0, NPAD - N), (0, 0)))
        h, at, n0, d0 = _stage_a(x_p, W, a_src, a_dst)
        w0, w1 = _sc_weights(src, dst, at)
        numf, denf = _sc_agg(a_src.shape[1], R, n_passes,
                             h, src, dst, w0, w1, n0, d0.reshape(-1))
        return _stage_c(numf[:N], denf.reshape(NPAD, 16)[:N], b, g, be)

    t = layer(x, W1, a_src1, a_dst1, b1, g1, be1, 160, 2)
    return layer(t, W2, a_src2, a_dst2, b2, g2, be2, 80, 4)


# speedup vs baseline: 7.5910x; 1.1861x over previous
import functools

import jax
import jax.numpy as jnp
from jax import lax
from jax.experimental import pallas as pl
from jax.experimental.pallas import tpu as pltpu
from jax.experimental.pallas import tpu_sc as plsc

HEADS = 2
N = 10000
E = 160000
NPAD = 10240
ROWBLK = 1280
EPT = E // 16
ECHUNK = 2000
NCHUNK = EPT // ECHUNK
CCAP = 2048
SUB = 16


def _stage_a_body(C, x_ref, w_ref, as_ref, ad_ref, h_ref, at_ref, n0_ref,
                  d0_ref):
    x = x_ref[...]
    h = jnp.dot(x, w_ref[...], preferred_element_type=jnp.float32)
    h0 = h[:, :C]
    h1 = h[:, C:]
    a_s = as_ref[...]
    a_d = ad_ref[...]
    asrc0 = jnp.sum(h0 * a_s[0:1, :], axis=1)
    asrc1 = jnp.sum(h1 * a_s[1:2, :], axis=1)
    adst0 = jnp.sum(h0 * a_d[0:1, :], axis=1)
    adst1 = jnp.sum(h1 * a_d[1:2, :], axis=1)
    s0 = asrc0 + adst0
    s1 = asrc1 + adst1
    ws0 = jnp.exp(jnp.where(s0 >= 0, s0, 0.2 * s0))
    ws1 = jnp.exp(jnp.where(s1 >= 0, s1, 0.2 * s1))
    h_ref[...] = h
    z = jnp.zeros((1, x.shape[0]), jnp.float32)
    at_ref[...] = jnp.concatenate(
        [asrc0[None], asrc1[None], adst0[None], adst1[None], z, z, z, z], 0)
    n0_ref[...] = jnp.concatenate([h0 * ws0[:, None], h1 * ws1[:, None]], 1)
    d0_ref[...] = jnp.concatenate(
        [ws0[:, None], ws1[:, None],
         jnp.zeros((x.shape[0], 14), jnp.float32)], 1)


def _stage_a(x_p, W, a_src, a_dst):
    cin = x_p.shape[1]
    C = a_src.shape[1]
    HC = HEADS * C
    nblk = NPAD // ROWBLK
    return pl.pallas_call(
        functools.partial(_stage_a_body, C),
        grid=(nblk,),
        in_specs=[
            pl.BlockSpec((ROWBLK, cin), lambda j: (j, 0)),
            pl.BlockSpec((cin, HC), lambda j: (0, 0)),
            pl.BlockSpec((HEADS, C), lambda j: (0, 0)),
            pl.BlockSpec((HEADS, C), lambda j: (0, 0)),
        ],
        out_specs=[
            pl.BlockSpec((ROWBLK, HC), lambda j: (j, 0)),
            pl.BlockSpec((8, ROWBLK), lambda j: (0, j)),
            pl.BlockSpec((ROWBLK, HC), lambda j: (j, 0)),
            pl.BlockSpec((ROWBLK, 16), lambda j: (j, 0)),
        ],
        out_shape=[
            jax.ShapeDtypeStruct((NPAD, HC), jnp.float32),
            jax.ShapeDtypeStruct((8, NPAD), jnp.float32),
            jax.ShapeDtypeStruct((NPAD, HC), jnp.float32),
            jax.ShapeDtypeStruct((NPAD, 16), jnp.float32),
        ],
    )(x_p, W, a_src, a_dst)


def _iota16():
    return lax.iota(jnp.int32, 16)


def _sc_w_body(srcr, dstr, alphat, w0o, w1o,
               as0_v, as1_v, ad0_v, ad1_v, src_v, dst_v, w0_v, w1_v):
    cid = lax.axis_index("c")
    sid = lax.axis_index("s")
    pltpu.sync_copy(alphat.at[0], as0_v)
    pltpu.sync_copy(alphat.at[1], as1_v)
    pltpu.sync_copy(alphat.at[2], ad0_v)
    pltpu.sync_copy(alphat.at[3], ad1_v)
    for c in range(NCHUNK):
        @pl.when((c % 2) == cid)
        def _():
            ebase = pl.multiple_of(sid * EPT + c * ECHUNK, 8)
            pltpu.sync_copy(srcr.at[pl.ds(ebase, ECHUNK)], src_v)
            pltpu.sync_copy(dstr.at[pl.ds(ebase, ECHUNK)], dst_v)

            def vb(i, carry):
                o = pl.multiple_of(i * 16, 16)
                s16 = src_v[pl.ds(o, 16)]
                d16 = dst_v[pl.ds(o, 16)]
                a0 = plsc.load_gather(as0_v, [s16])
                a1 = plsc.load_gather(as1_v, [s16])
                b0 = plsc.load_gather(ad0_v, [d16])
                b1 = plsc.load_gather(ad1_v, [d16])
                e0 = a0 + b0
                e1 = a1 + b1
                w0_v[pl.ds(o, 16)] = jnp.exp(
                    jnp.where(e0 >= 0, e0, 0.2 * e0))
                w1_v[pl.ds(o, 16)] = jnp.exp(
                    jnp.where(e1 >= 0, e1, 0.2 * e1))
                return carry

            lax.fori_loop(0, ECHUNK // 16, vb, 0)
            pltpu.sync_copy(w0_v, w0o.at[pl.ds(ebase, ECHUNK)])
            pltpu.sync_copy(w1_v, w1o.at[pl.ds(ebase, ECHUNK)])


def _sc_weights(src, dst, alphat):
    mesh = plsc.VectorSubcoreMesh(core_axis_name="c", subcore_axis_name="s")
    f = pl.kernel(
        _sc_w_body,
        mesh=mesh,
        compiler_params=pltpu.CompilerParams(needs_layout_passes=False),
        out_type=[
            jax.ShapeDtypeStruct((E,), jnp.float32),
            jax.ShapeDtypeStruct((E,), jnp.float32),
        ],
        scratch_types=[
            pltpu.VMEM((NPAD,), jnp.float32),
            pltpu.VMEM((NPAD,), jnp.float32),
            pltpu.VMEM((NPAD,), jnp.float32),
            pltpu.VMEM((NPAD,), jnp.float32),
            pltpu.VMEM((ECHUNK,), jnp.int32),
            pltpu.VMEM((ECHUNK,), jnp.int32),
            pltpu.VMEM((ECHUNK,), jnp.float32),
            pltpu.VMEM((ECHUNK,), jnp.float32),
        ],
    )
    return f(src, dst, alphat)


def _sc_agg_body(C, R, n_passes,
                 h, srcr, dstr, w0r, w1r, num0, den0, numf, denf,
                 src_a, dst_a, w0a, w1a, src_b, dst_b, w0b, w1b,
                 csrc, cdl, cw0, cw1, rows_v, gidx,
                 accf, accd, sem_a, sem_b):
    HC = HEADS * C
    cid = lax.axis_index("c")
    sid = lax.axis_index("s")
    wid = sid * 2 + cid
    zc16 = jnp.zeros((16,), jnp.int32)
    zf16 = jnp.zeros((16,), jnp.float32)
    iota = _iota16()

    bufs_a = (src_a, dst_a, w0a, w1a)
    bufs_b = (src_b, dst_b, w0b, w1b)
    hbm = (srcr, dstr, w0r, w1r)

    def start_chunk(chunk, bufs, sem):
        ebase = pl.multiple_of(chunk * ECHUNK, 8)
        for hr, vr in zip(hbm, bufs):
            pltpu.async_copy(hr.at[pl.ds(ebase, ECHUNK)], vr, sem)

    def wait_chunk(bufs, sem):
        for vr in bufs:
            pltpu.make_async_copy(srcr.at[pl.ds(0, ECHUNK)], vr, sem).wait()

    for p in range(n_passes):
        lo = (p * 32 + wid) * R
        pltpu.sync_copy(num0.at[pl.ds(lo, R)], accf)
        pltpu.sync_copy(den0.at[pl.ds(pl.multiple_of(lo * 16, 16), R * 16)],
                        accd)

        def process(src_v, dst_v, w0c, w1c):
            def vbody(i, off):
                o = pl.multiple_of(i * 16, 16)
                s16 = src_v[pl.ds(o, 16)]
                d16 = dst_v[pl.ds(o, 16)]
                w0 = w0c[pl.ds(o, 16)]
                w1 = w1c[pl.ds(o, 16)]
                dl = d16 - lo
                m = (dl >= 0) & (dl < R)
                mi = m.astype(jnp.int32)
                pos = off + plsc.cumsum(mi) - mi
                plsc.store_scatter(csrc, [pos], s16, mask=m)
                plsc.store_scatter(cdl, [pos], dl, mask=m)
                plsc.store_scatter(cw0, [pos], w0, mask=m)
                plsc.store_scatter(cw1, [pos], w1, mask=m)
                return off + jnp.sum(mi)

            off = lax.fori_loop(0, ECHUNK // 16, vbody, jnp.int32(0))

            nsub = (off + (SUB - 1)) // SUB
            pad_end = nsub * SUB
            for k in range(SUB // 16):
                idx = off + k * 16 + iota
                pm = idx < pad_end
                plsc.store_scatter(csrc, [idx], zc16, mask=pm)
                plsc.store_scatter(cdl, [idx], zc16, mask=pm)
                plsc.store_scatter(cw0, [idx], zf16, mask=pm)
                plsc.store_scatter(cw1, [idx], zf16, mask=pm)

            def drain(j, carry):
                for k in range(SUB // 16):
                    ok = pl.multiple_of(j * SUB + k * 16, 16)
                    gidx[pl.ds(k * 16, 16)] = csrc[pl.ds(ok, 16)]
                pltpu.sync_copy(h.at[gidx], rows_v)

                def srow(r, c2):
                    i = j * SUB + r
                    si = jnp.full((16,), i, jnp.int32)
                    rs = jnp.full((16,), r, jnp.int32)
                    dlv = plsc.load_gather(cdl, [si])
                    f0 = plsc.load_gather(cw0, [si])
                    f1 = plsc.load_gather(cw1, [si])
                    wrow = jnp.where(iota == 0, f0,
                                     jnp.where(iota == 1, f1, zf16))
                    plsc.addupdate_scatter(accd, [dlv * 16 + iota], wrow)
                    for q in range(HC // 16):
                        cq = q * 16 + iota
                        f = f0 if q < (C // 16) else f1
                        v = plsc.load_gather(rows_v, [rs, cq]) * f
                        plsc.addupdate_scatter(accf, [dlv, cq], v)
                    return c2

                lax.fori_loop(0, SUB, srow, 0)
                return carry

            lax.fori_loop(0, nsub, drain, 0)

        nchunks = E // ECHUNK

        def cpair(c2, carry):
            ca = 2 * c2
            start_chunk(ca + 1, bufs_b, sem_b)
            wait_chunk(bufs_a, sem_a)
            process(*bufs_a)

            @pl.when(ca + 2 < nchunks)
            def _():
                start_chunk(ca + 2, bufs_a, sem_a)

            wait_chunk(bufs_b, sem_b)
            process(*bufs_b)
            return carry

        start_chunk(0, bufs_a, sem_a)
        lax.fori_loop(0, nchunks // 2, cpair, 0)

        pltpu.sync_copy(accf, numf.at[pl.ds(lo, R)])
        pltpu.sync_copy(accd,
                        denf.at[pl.ds(pl.multiple_of(lo * 16, 16), R * 16)])


def _sc_agg(C, R, n_passes, h, src, dst, w0, w1, num0, den0):
    HC = HEADS * C
    mesh = plsc.VectorSubcoreMesh(core_axis_name="c", subcore_axis_name="s")
    f = pl.kernel(
        functools.partial(_sc_agg_body, C, R, n_passes),
        mesh=mesh,
        compiler_params=pltpu.CompilerParams(needs_layout_passes=False),
        out_type=[
            jax.ShapeDtypeStruct((NPAD, HC), jnp.float32),
            jax.ShapeDtypeStruct((NPAD * 16,), jnp.float32),
        ],
        scratch_types=[
            pltpu.VMEM((ECHUNK,), jnp.int32),
            pltpu.VMEM((ECHUNK,), jnp.int32),
            pltpu.VMEM((ECHUNK,), jnp.float32),
            pltpu.VMEM((ECHUNK,), jnp.float32),
            pltpu.VMEM((ECHUNK,), jnp.int32),
            pltpu.VMEM((ECHUNK,), jnp.int32),
            pltpu.VMEM((ECHUNK,), jnp.float32),
            pltpu.VMEM((ECHUNK,), jnp.float32),
            pltpu.VMEM((CCAP,), jnp.int32),
            pltpu.VMEM((CCAP,), jnp.int32),
            pltpu.VMEM((CCAP,), jnp.float32),
            pltpu.VMEM((CCAP,), jnp.float32),
            pltpu.VMEM((SUB, HC), jnp.float32),
            pltpu.VMEM((SUB,), jnp.int32),
            pltpu.VMEM((R, HC), jnp.float32),
            pltpu.VMEM((R * 16,), jnp.float32),
            pltpu.SemaphoreType.DMA,
            pltpu.SemaphoreType.DMA,
        ],
    )
    return f(h, src, dst, w0, w1, num0, den0)


def _stage_c_body(nbh, num_ref, den_ref, b_ref, g_ref, be_ref, o_ref):
    j = pl.program_id(0)
    num = num_ref[...]
    den = den_ref[...]
    d = jnp.where(j < nbh, den[:, 0:1], den[:, 1:2])
    pre = num / (d + 1e-16) + b_ref[...]
    mu = jnp.mean(pre, axis=0, keepdims=True)
    var = jnp.mean((pre - mu) ** 2, axis=0, keepdims=True)
    y = (pre - mu) * lax.rsqrt(var + 1e-5) * g_ref[...] + be_ref[...]
    o_ref[...] = jnp.maximum(y, 0.0)


def _stage_c(numf, denf, b, g, be):
    HC = b.shape[0]
    C = HC // HEADS
    nblk = HC // 128
    return pl.pallas_call(
        functools.partial(_stage_c_body, C // 128),
        grid=(nblk,),
        in_specs=[
            pl.BlockSpec((N, 128), lambda j: (0, j)),
            pl.BlockSpec((N, 16), lambda j: (0, 0)),
            pl.BlockSpec((1, 128), lambda j: (0, j)),
            pl.BlockSpec((1, 128), lambda j: (0, j)),
            pl.BlockSpec((1, 128), lambda j: (0, j)),
        ],
        out_specs=pl.BlockSpec((N, 128), lambda j: (0, j)),
        out_shape=jax.ShapeDtypeStruct((N, HC), jnp.float32),
    )(numf, denf, b.reshape(1, HC), g.reshape(1, HC), be.reshape(1, HC))


def kernel(x, edge_index, W1, a_src1, a_dst1, b1, g1, be1,
           W2, a_src2, a_dst2, b2, g2, be2):
    src = edge_index[0].astype(jnp.int32)
    dst = edge_index[1].astype(jnp.int32)

    def layer(x_in, W, a_src, a_dst, b, g, be, R, n_passes):
        x_p = jnp.pad(x_in, ((0, NPAD - N), (0, 0)))
        h, at, n0, d0 = _stage_a(x_p, W, a_src, a_dst)
        w0, w1 = _sc_weights(src, dst, at)
        numf, denf = _sc_agg(a_src.shape[1], R, n_passes,
                             h, src, dst, w0, w1, n0, d0.reshape(-1))
        return _stage_c(numf[:N], denf.reshape(NPAD, 16)[:N], b, g, be)

    t = layer(x, W1, a_src1, a_dst1, b1, g1, be1, 160, 2)
    return layer(t, W2, a_src2, a_dst2, b2, g2, be2, 80, 4)


# speedup vs baseline: 10.8520x; 1.4296x over previous
import functools

import jax
import jax.numpy as jnp
from jax import lax
from jax.experimental import pallas as pl
from jax.experimental.pallas import tpu as pltpu
from jax.experimental.pallas import tpu_sc as plsc

HEADS = 2
N = 10000
E = 160000
NPAD = 10240
ROWBLK = 1280
EPT = E // 16
NBKT = 32
BROWS = NPAD // NBKT
MULT, SHR = 6554, 21
SEGCAP = 10240
SEGTOT = 16 * NBKT * SEGCAP
WCHUNK = 2000
ACHUNK = 512
ACCAP = ACHUNK + 16
SUB = 16


def _stage_a_body(C, x_ref, w_ref, as_ref, ad_ref, h_ref, at_ref, n0_ref,
                  d0_ref):
    x = x_ref[...]
    h = jnp.dot(x, w_ref[...], preferred_element_type=jnp.float32)
    h0 = h[:, :C]
    h1 = h[:, C:]
    a_s = as_ref[...]
    a_d = ad_ref[...]
    asrc0 = jnp.sum(h0 * a_s[0:1, :], axis=1)
    asrc1 = jnp.sum(h1 * a_s[1:2, :], axis=1)
    adst0 = jnp.sum(h0 * a_d[0:1, :], axis=1)
    adst1 = jnp.sum(h1 * a_d[1:2, :], axis=1)
    s0 = asrc0 + adst0
    s1 = asrc1 + adst1
    ws0 = jnp.exp(jnp.where(s0 >= 0, s0, 0.2 * s0))
    ws1 = jnp.exp(jnp.where(s1 >= 0, s1, 0.2 * s1))
    h_ref[...] = h
    z = jnp.zeros((1, x.shape[0]), jnp.float32)
    at_ref[...] = jnp.concatenate(
        [asrc0[None], asrc1[None], adst0[None], adst1[None], z, z, z, z], 0)
    n0_ref[...] = jnp.concatenate([h0 * ws0[:, None], h1 * ws1[:, None]], 1)
    d0_ref[...] = jnp.concatenate(
        [ws0[:, None], ws1[:, None],
         jnp.zeros((x.shape[0], 14), jnp.float32)], 1)


def _stage_a(x_p, W, a_src, a_dst):
    cin = x_p.shape[1]
    C = a_src.shape[1]
    HC = HEADS * C
    nblk = NPAD // ROWBLK
    return pl.pallas_call(
        functools.partial(_stage_a_body, C),
        grid=(nblk,),
        in_specs=[
            pl.BlockSpec((ROWBLK, cin), lambda j: (j, 0)),
            pl.BlockSpec((cin, HC), lambda j: (0, 0)),
            pl.BlockSpec((HEADS, C), lambda j: (0, 0)),
            pl.BlockSpec((HEADS, C), lambda j: (0, 0)),
        ],
        out_specs=[
            pl.BlockSpec((ROWBLK, HC), lambda j: (j, 0)),
            pl.BlockSpec((8, ROWBLK), lambda j: (0, j)),
            pl.BlockSpec((ROWBLK, HC), lambda j: (j, 0)),
            pl.BlockSpec((ROWBLK, 16), lambda j: (j, 0)),
        ],
        out_shape=[
            jax.ShapeDtypeStruct((NPAD, HC), jnp.float32),
            jax.ShapeDtypeStruct((8, NPAD), jnp.float32),
            jax.ShapeDtypeStruct((NPAD, HC), jnp.float32),
            jax.ShapeDtypeStruct((NPAD, 16), jnp.float32),
        ],
    )(x_p, W, a_src, a_dst)


def _iota16():
    return lax.iota(jnp.int32, 16)


def _sc_bin_body(srcr, dstr, src_o, dst_o, cnt_o,
                 src_v, dst_v, cs, cd, cnt_v):
    cid = lax.axis_index("c")
    sid = lax.axis_index("s")
    iota = _iota16()
    ebase = pl.multiple_of(sid * EPT, 8)
    pltpu.sync_copy(srcr.at[pl.ds(ebase, EPT)], src_v)
    pltpu.sync_copy(dstr.at[pl.ds(ebase, EPT)], dst_v)
    for bl in range(16):
        b = bl + cid * 16

        def vbody(i, off):
            o = pl.multiple_of(i * 16, 16)
            s16 = src_v[pl.ds(o, 16)]
            d16 = dst_v[pl.ds(o, 16)]
            bk = lax.shift_right_logical(d16 * MULT, SHR)
            m = bk == b
            mi = m.astype(jnp.int32)
            pos = off + plsc.cumsum(mi) - mi
            plsc.store_scatter(cs, [pos], s16, mask=m)
            plsc.store_scatter(cd, [pos], d16, mask=m)
            return off + jnp.sum(mi)

        off = lax.fori_loop(0, EPT // 16, vbody, jnp.int32(0))
        plsc.store_scatter(cnt_v, [jnp.full((16,), bl, jnp.int32)],
                           jnp.full((16,), off, jnp.int32),
                           mask=iota == 0)
        seg = pl.multiple_of((sid * NBKT + b) * SEGCAP, 8)
        pltpu.sync_copy(cs, src_o.at[pl.ds(seg, SEGCAP)])
        pltpu.sync_copy(cd, dst_o.at[pl.ds(seg, SEGCAP)])
    pltpu.sync_copy(cnt_v,
                    cnt_o.at[pl.ds(pl.multiple_of(
                        sid * NBKT + cid * 16, 8), 16)])


def _sc_bin(src, dst):
    mesh = plsc.VectorSubcoreMesh(core_axis_name="c", subcore_axis_name="s")
    f = pl.kernel(
        _sc_bin_body,
        mesh=mesh,
        compiler_params=pltpu.CompilerParams(needs_layout_passes=False),
        out_type=[
            jax.ShapeDtypeStruct((SEGTOT,), jnp.int32),
            jax.ShapeDtypeStruct((SEGTOT,), jnp.int32),
            jax.ShapeDtypeStruct((16 * NBKT,), jnp.int32),
        ],
        scratch_types=[
            pltpu.VMEM((EPT,), jnp.int32),
            pltpu.VMEM((EPT,), jnp.int32),
            pltpu.VMEM((SEGCAP,), jnp.int32),
            pltpu.VMEM((SEGCAP,), jnp.int32),
            pltpu.VMEM((16,), jnp.int32),
        ],
    )
    return f(src, dst)


def _sc_w_body(srcb, dstb, cnt_o, alphat, w0o, w1o,
               as0_v, as1_v, ad0_v, ad1_v, cnt_v, src_c, dst_c, w0_c, w1_c):
    cid = lax.axis_index("c")
    sid = lax.axis_index("s")
    pltpu.sync_copy(alphat.at[0], as0_v)
    pltpu.sync_copy(alphat.at[1], as1_v)
    pltpu.sync_copy(alphat.at[2], ad0_v)
    pltpu.sync_copy(alphat.at[3], ad1_v)
    pltpu.sync_copy(cnt_o.at[pl.ds(pl.multiple_of(
        sid * NBKT + cid * 16, 8), 16)], cnt_v)
    for bl in range(16):
        b = bl + cid * 16
        cnt = jnp.max(plsc.load_gather(
            cnt_v, [jnp.full((16,), bl, jnp.int32)]))
        seg = (sid * NBKT + b) * SEGCAP
        nch = (cnt + (WCHUNK - 1)) // WCHUNK

        def cbody(c, carry):
            base = pl.multiple_of(seg + c * WCHUNK, 8)
            pltpu.sync_copy(srcb.at[pl.ds(base, WCHUNK)], src_c)
            pltpu.sync_copy(dstb.at[pl.ds(base, WCHUNK)], dst_c)

            def vb(i, carry2):
                o = pl.multiple_of(i * 16, 16)
                s16 = jnp.clip(src_c[pl.ds(o, 16)], 0, NPAD - 1)
                d16 = jnp.clip(dst_c[pl.ds(o, 16)], 0, NPAD - 1)
                a0 = plsc.load_gather(as0_v, [s16])
                a1 = plsc.load_gather(as1_v, [s16])
                b0 = plsc.load_gather(ad0_v, [d16])
                b1 = plsc.load_gather(ad1_v, [d16])
                e0 = a0 + b0
                e1 = a1 + b1
                w0_c[pl.ds(o, 16)] = jnp.exp(
                    jnp.where(e0 >= 0, e0, 0.2 * e0))
                w1_c[pl.ds(o, 16)] = jnp.exp(
                    jnp.where(e1 >= 0, e1, 0.2 * e1))
                return carry2

            lax.fori_loop(0, WCHUNK // 16, vb, 0)
            pltpu.sync_copy(w0_c, w0o.at[pl.ds(base, WCHUNK)])
            pltpu.sync_copy(w1_c, w1o.at[pl.ds(base, WCHUNK)])
            return carry

        lax.fori_loop(0, nch, cbody, 0)


def _sc_weights(srcb, dstb, cnts, alphat):
    mesh = plsc.VectorSubcoreMesh(core_axis_name="c", subcore_axis_name="s")
    f = pl.kernel(
        _sc_w_body,
        mesh=mesh,
        compiler_params=pltpu.CompilerParams(needs_layout_passes=False),
        out_type=[
            jax.ShapeDtypeStruct((SEGTOT,), jnp.float32),
            jax.ShapeDtypeStruct((SEGTOT,), jnp.float32),
        ],
        scratch_types=[
            pltpu.VMEM((NPAD,), jnp.float32),
            pltpu.VMEM((NPAD,), jnp.float32),
            pltpu.VMEM((NPAD,), jnp.float32),
            pltpu.VMEM((NPAD,), jnp.float32),
            pltpu.VMEM((16,), jnp.int32),
            pltpu.VMEM((WCHUNK,), jnp.int32),
            pltpu.VMEM((WCHUNK,), jnp.int32),
            pltpu.VMEM((WCHUNK,), jnp.float32),
            pltpu.VMEM((WCHUNK,), jnp.float32),
        ],
    )
    return f(srcb, dstb, cnts, alphat)


def _sc_agg_body(C, R, n_passes,
                 h, srcb, dstb, w0r, w1r, cnt_o, num0, den0, numf, denf,
                 cnts_v, src_v, dst_v, w0c, w1c,
                 csrc, cdl, cw0, cw1, rows_v, gidx,
                 accf, accd):
    HC = HEADS * C
    cid = lax.axis_index("c")
    sid = lax.axis_index("s")
    wid = sid * 2 + cid
    zc16 = jnp.zeros((16,), jnp.int32)
    zf16 = jnp.zeros((16,), jnp.float32)
    iota = _iota16()

    pltpu.sync_copy(cnt_o, cnts_v)

    for p in range(n_passes):
        lo = (p * 32 + wid) * R
        b = lo // BROWS
        pltpu.sync_copy(num0.at[pl.ds(lo, R)], accf)
        pltpu.sync_copy(den0.at[pl.ds(pl.multiple_of(lo * 16, 16), R * 16)],
                        accd)

        def wbody(w, wcarry):
            sb = w * NBKT + b
            cnt = jnp.max(plsc.load_gather(
                cnts_v, [jnp.full((16,), sb, jnp.int32)]))
            seg = sb * SEGCAP
            nch = (cnt + (ACHUNK - 1)) // ACHUNK

            def cbody(c, carry0):
                cb = c * ACHUNK
                base = pl.multiple_of(seg + cb, 8)
                pltpu.sync_copy(srcb.at[pl.ds(base, ACHUNK)], src_v)
                pltpu.sync_copy(dstb.at[pl.ds(base, ACHUNK)], dst_v)
                pltpu.sync_copy(w0r.at[pl.ds(base, ACHUNK)], w0c)
                pltpu.sync_copy(w1r.at[pl.ds(base, ACHUNK)], w1c)
                rem = cnt - cb
                nv = jnp.minimum(ACHUNK // 16, (rem + 15) // 16)

                def vbody(i, off):
                    o = pl.multiple_of(i * 16, 16)
                    s16 = src_v[pl.ds(o, 16)]
                    d16 = dst_v[pl.ds(o, 16)]
                    w0 = w0c[pl.ds(o, 16)]
                    w1 = w1c[pl.ds(o, 16)]
                    dl = d16 - lo
                    m = ((dl >= 0) & (dl < R)) & (cb + o + iota < cnt)
                    mi = m.astype(jnp.int32)
                    pos = off + plsc.cumsum(mi) - mi
                    plsc.store_scatter(csrc, [pos], s16, mask=m)
                    plsc.store_scatter(cdl, [pos], dl, mask=m)
                    plsc.store_scatter(cw0, [pos], w0, mask=m)
                    plsc.store_scatter(cw1, [pos], w1, mask=m)
                    return off + jnp.sum(mi)

                off = lax.fori_loop(0, nv, vbody, jnp.int32(0))

                nsub = (off + (SUB - 1)) // SUB
                pad_end = nsub * SUB
                for k in range(SUB // 16):
                    idx = off + k * 16 + iota
                    pm = idx < pad_end
                    plsc.store_scatter(csrc, [idx], zc16, mask=pm)
                    plsc.store_scatter(cdl, [idx], zc16, mask=pm)
                    plsc.store_scatter(cw0, [idx], zf16, mask=pm)
                    plsc.store_scatter(cw1, [idx], zf16, mask=pm)

                def drain(j, carry):
                    for k in range(SUB // 16):
                        ok = pl.multiple_of(j * SUB + k * 16, 16)
                        gidx[pl.ds(k * 16, 16)] = csrc[pl.ds(ok, 16)]
                    pltpu.sync_copy(h.at[gidx], rows_v)

                    def srow(r, c2):
                        i = j * SUB + r
                        si = jnp.full((16,), i, jnp.int32)
                        rs = jnp.full((16,), r, jnp.int32)
                        dlv = plsc.load_gather(cdl, [si])
                        f0 = plsc.load_gather(cw0, [si])
                        f1 = plsc.load_gather(cw1, [si])
                        wrow = jnp.where(iota == 0, f0,
                                         jnp.where(iota == 1, f1, zf16))
                        plsc.addupdate_scatter(accd, [dlv * 16 + iota], wrow)
                        for q in range(HC // 16):
                            cq = q * 16 + iota
                            f = f0 if q < (C // 16) else f1
                            v = plsc.load_gather(rows_v, [rs, cq]) * f
                            plsc.addupdate_scatter(accf, [dlv, cq], v)
                        return c2

                    lax.fori_loop(0, SUB, srow, 0)
                    return carry

                lax.fori_loop(0, nsub, drain, 0)
                return carry0

            lax.fori_loop(0, nch, cbody, 0)
            return wcarry

        lax.fori_loop(0, 16, wbody, 0)

        pltpu.sync_copy(accf, numf.at[pl.ds(lo, R)])
        pltpu.sync_copy(accd,
                        denf.at[pl.ds(pl.multiple_of(lo * 16, 16), R * 16)])


def _sc_agg(C, R, n_passes, h, srcb, dstb, w0, w1, cnts, num0, den0):
    HC = HEADS * C
    mesh = plsc.VectorSubcoreMesh(core_axis_name="c", subcore_axis_name="s")
    f = pl.kernel(
        functools.partial(_sc_agg_body, C, R, n_passes),
        mesh=mesh,
        compiler_params=pltpu.CompilerParams(needs_layout_passes=False),
        out_type=[
            jax.ShapeDtypeStruct((NPAD, HC), jnp.float32),
            jax.ShapeDtypeStruct((NPAD * 16,), jnp.float32),
        ],
        scratch_types=[
            pltpu.VMEM((16 * NBKT,), jnp.int32),
            pltpu.VMEM((ACHUNK,), jnp.int32),
            pltpu.VMEM((ACHUNK,), jnp.int32),
            pltpu.VMEM((ACHUNK,), jnp.float32),
            pltpu.VMEM((ACHUNK,), jnp.float32),
            pltpu.VMEM((ACCAP,), jnp.int32),
            pltpu.VMEM((ACCAP,), jnp.int32),
            pltpu.VMEM((ACCAP,), jnp.float32),
            pltpu.VMEM((ACCAP,), jnp.float32),
            pltpu.VMEM((SUB, HC), jnp.float32),
            pltpu.VMEM((SUB,), jnp.int32),
            pltpu.VMEM((R, HC), jnp.float32),
            pltpu.VMEM((R * 16,), jnp.float32),
        ],
    )
    return f(h, srcb, dstb, w0, w1, cnts, num0, den0)


def _stage_c_body(nbh, num_ref, den_ref, b_ref, g_ref, be_ref, o_ref):
    j = pl.program_id(0)
    num = num_ref[...]
    den = den_ref[...]
    d = jnp.where(j < nbh, den[:, 0:1], den[:, 1:2])
    pre = num / (d + 1e-16) + b_ref[...]
    mu = jnp.mean(pre, axis=0, keepdims=True)
    var = jnp.mean((pre - mu) ** 2, axis=0, keepdims=True)
    y = (pre - mu) * lax.rsqrt(var + 1e-5) * g_ref[...] + be_ref[...]
    o_ref[...] = jnp.maximum(y, 0.0)


def _stage_c(numf, denf, b, g, be):
    HC = b.shape[0]
    C = HC // HEADS
    nblk = HC // 128
    return pl.pallas_call(
        functools.partial(_stage_c_body, C // 128),
        grid=(nblk,),
        in_specs=[
            pl.BlockSpec((N, 128), lambda j: (0, j)),
            pl.BlockSpec((N, 16), lambda j: (0, 0)),
            pl.BlockSpec((1, 128), lambda j: (0, j)),
            pl.BlockSpec((1, 128), lambda j: (0, j)),
            pl.BlockSpec((1, 128), lambda j: (0, j)),
        ],
        out_specs=pl.BlockSpec((N, 128), lambda j: (0, j)),
        out_shape=jax.ShapeDtypeStruct((N, HC), jnp.float32),
    )(numf, denf, b.reshape(1, HC), g.reshape(1, HC), be.reshape(1, HC))


def kernel(x, edge_index, W1, a_src1, a_dst1, b1, g1, be1,
           W2, a_src2, a_dst2, b2, g2, be2):
    src = edge_index[0].astype(jnp.int32)
    dst = edge_index[1].astype(jnp.int32)
    srcb, dstb, cnts = _sc_bin(src, dst)

    def layer(x_in, W, a_src, a_dst, b, g, be, R, n_passes):
        x_p = jnp.pad(x_in, ((0, NPAD - N), (0, 0)))
        h, at, n0, d0 = _stage_a(x_p, W, a_src, a_dst)
        w0, w1 = _sc_weights(srcb, dstb, cnts, at)
        numf, denf = _sc_agg(a_src.shape[1], R, n_passes,
                             h, srcb, dstb, w0, w1, cnts, n0,
                             d0.reshape(-1))
        return _stage_c(numf[:N], denf.reshape(NPAD, 16)[:N], b, g, be)

    t = layer(x, W1, a_src1, a_dst1, b1, g1, be1, 160, 2)
    return layer(t, W2, a_src2, a_dst2, b2, g2, be2, 80, 4)


# speedup vs baseline: 12.9282x; 1.1913x over previous
import functools

import jax
import jax.numpy as jnp
from jax import lax
from jax.experimental import pallas as pl
from jax.experimental.pallas import tpu as pltpu
from jax.experimental.pallas import tpu_sc as plsc

HEADS = 2
N = 10000
E = 160000
NPAD = 10240
ROWBLK = 1280
EPT = E // 16
NBKT = 32
BROWS = NPAD // NBKT
MULT, SHR = 6554, 21
SEGCAP = 10240
SEGTOT = 16 * NBKT * SEGCAP
WCHUNK = 2000
ACHUNK = 512
ACCAP = ACHUNK + 16
SUB = 16


def _stage_a_body(C, x_ref, w_ref, as_ref, ad_ref, h_ref, at_ref, n0_ref,
                  d0_ref):
    x = x_ref[...]
    h = jnp.dot(x, w_ref[...], preferred_element_type=jnp.float32)
    h0 = h[:, :C]
    h1 = h[:, C:]
    a_s = as_ref[...]
    a_d = ad_ref[...]
    asrc0 = jnp.sum(h0 * a_s[0:1, :], axis=1)
    asrc1 = jnp.sum(h1 * a_s[1:2, :], axis=1)
    adst0 = jnp.sum(h0 * a_d[0:1, :], axis=1)
    adst1 = jnp.sum(h1 * a_d[1:2, :], axis=1)
    s0 = asrc0 + adst0
    s1 = asrc1 + adst1
    ws0 = jnp.exp(jnp.where(s0 >= 0, s0, 0.2 * s0))
    ws1 = jnp.exp(jnp.where(s1 >= 0, s1, 0.2 * s1))
    h_ref[...] = h
    z = jnp.zeros((1, x.shape[0]), jnp.float32)
    at_ref[...] = jnp.concatenate(
        [asrc0[None], asrc1[None], adst0[None], adst1[None], z, z, z, z], 0)
    n0_ref[...] = jnp.concatenate([h0 * ws0[:, None], h1 * ws1[:, None]], 1)
    d0_ref[...] = jnp.concatenate(
        [ws0[:, None], ws1[:, None],
         jnp.zeros((x.shape[0], 14), jnp.float32)], 1)


def _stage_a(x_p, W, a_src, a_dst):
    cin = x_p.shape[1]
    C = a_src.shape[1]
    HC = HEADS * C
    nblk = NPAD // ROWBLK
    return pl.pallas_call(
        functools.partial(_stage_a_body, C),
        grid=(nblk,),
        in_specs=[
            pl.BlockSpec((ROWBLK, cin), lambda j: (j, 0)),
            pl.BlockSpec((cin, HC), lambda j: (0, 0)),
            pl.BlockSpec((HEADS, C), lambda j: (0, 0)),
            pl.BlockSpec((HEADS, C), lambda j: (0, 0)),
        ],
        out_specs=[
            pl.BlockSpec((ROWBLK, HC), lambda j: (j, 0)),
            pl.BlockSpec((8, ROWBLK), lambda j: (0, j)),
            pl.BlockSpec((ROWBLK, HC), lambda j: (j, 0)),
            pl.BlockSpec((ROWBLK, 16), lambda j: (j, 0)),
        ],
        out_shape=[
            jax.ShapeDtypeStruct((NPAD, HC), jnp.float32),
            jax.ShapeDtypeStruct((8, NPAD), jnp.float32),
            jax.ShapeDtypeStruct((NPAD, HC), jnp.float32),
            jax.ShapeDtypeStruct((NPAD, 16), jnp.float32),
        ],
    )(x_p, W, a_src, a_dst)


def _iota16():
    return lax.iota(jnp.int32, 16)


def _sc_bin_body(srcr, dstr, src_o, dst_o, cnt_o,
                 src_v, dst_v, cs, cd, cnt_v):
    cid = lax.axis_index("c")
    sid = lax.axis_index("s")
    iota = _iota16()
    ebase = pl.multiple_of(sid * EPT, 8)
    pltpu.sync_copy(srcr.at[pl.ds(ebase, EPT)], src_v)
    pltpu.sync_copy(dstr.at[pl.ds(ebase, EPT)], dst_v)
    for bl in range(16):
        b = bl + cid * 16

        def vbody(i, off):
            o = pl.multiple_of(i * 16, 16)
            s16 = src_v[pl.ds(o, 16)]
            d16 = dst_v[pl.ds(o, 16)]
            bk = lax.shift_right_logical(d16 * MULT, SHR)
            m = bk == b
            mi = m.astype(jnp.int32)
            pos = off + plsc.cumsum(mi) - mi
            plsc.store_scatter(cs, [pos], s16, mask=m)
            plsc.store_scatter(cd, [pos], d16, mask=m)
            return off + jnp.sum(mi)

        off = lax.fori_loop(0, EPT // 16, vbody, jnp.int32(0))
        plsc.store_scatter(cnt_v, [jnp.full((16,), bl, jnp.int32)],
                           jnp.full((16,), off, jnp.int32),
                           mask=iota == 0)
        seg = pl.multiple_of((sid * NBKT + b) * SEGCAP, 8)
        pltpu.sync_copy(cs, src_o.at[pl.ds(seg, SEGCAP)])
        pltpu.sync_copy(cd, dst_o.at[pl.ds(seg, SEGCAP)])
    pltpu.sync_copy(cnt_v,
                    cnt_o.at[pl.ds(pl.multiple_of(
                        sid * NBKT + cid * 16, 8), 16)])


def _sc_bin(src, dst):
    mesh = plsc.VectorSubcoreMesh(core_axis_name="c", subcore_axis_name="s")
    f = pl.kernel(
        _sc_bin_body,
        mesh=mesh,
        compiler_params=pltpu.CompilerParams(needs_layout_passes=False),
        out_type=[
            jax.ShapeDtypeStruct((SEGTOT,), jnp.int32),
            jax.ShapeDtypeStruct((SEGTOT,), jnp.int32),
            jax.ShapeDtypeStruct((16 * NBKT,), jnp.int32),
        ],
        scratch_types=[
            pltpu.VMEM((EPT,), jnp.int32),
            pltpu.VMEM((EPT,), jnp.int32),
            pltpu.VMEM((SEGCAP,), jnp.int32),
            pltpu.VMEM((SEGCAP,), jnp.int32),
            pltpu.VMEM((16,), jnp.int32),
        ],
    )
    return f(src, dst)


def _sc_w_body(srcb, dstb, cnt_o, alphat, w0o, w1o,
               as0_v, as1_v, ad0_v, ad1_v, cnt_v, src_c, dst_c, w0_c, w1_c):
    cid = lax.axis_index("c")
    sid = lax.axis_index("s")
    pltpu.sync_copy(alphat.at[0], as0_v)
    pltpu.sync_copy(alphat.at[1], as1_v)
    pltpu.sync_copy(alphat.at[2], ad0_v)
    pltpu.sync_copy(alphat.at[3], ad1_v)
    pltpu.sync_copy(cnt_o.at[pl.ds(pl.multiple_of(
        sid * NBKT + cid * 16, 8), 16)], cnt_v)
    for bl in range(16):
        b = bl + cid * 16
        cnt = jnp.max(plsc.load_gather(
            cnt_v, [jnp.full((16,), bl, jnp.int32)]))
        seg = (sid * NBKT + b) * SEGCAP
        nch = (cnt + (WCHUNK - 1)) // WCHUNK

        def cbody(c, carry):
            base = pl.multiple_of(seg + c * WCHUNK, 8)
            pltpu.sync_copy(srcb.at[pl.ds(base, WCHUNK)], src_c)
            pltpu.sync_copy(dstb.at[pl.ds(base, WCHUNK)], dst_c)

            def vb(i, carry2):
                o = pl.multiple_of(i * 16, 16)
                s16 = jnp.clip(src_c[pl.ds(o, 16)], 0, NPAD - 1)
                d16 = jnp.clip(dst_c[pl.ds(o, 16)], 0, NPAD - 1)
                a0 = plsc.load_gather(as0_v, [s16])
                a1 = plsc.load_gather(as1_v, [s16])
                b0 = plsc.load_gather(ad0_v, [d16])
                b1 = plsc.load_gather(ad1_v, [d16])
                e0 = a0 + b0
                e1 = a1 + b1
                w0_c[pl.ds(o, 16)] = jnp.exp(
                    jnp.where(e0 >= 0, e0, 0.2 * e0))
                w1_c[pl.ds(o, 16)] = jnp.exp(
                    jnp.where(e1 >= 0, e1, 0.2 * e1))
                return carry2

            lax.fori_loop(0, WCHUNK // 16, vb, 0)
            pltpu.sync_copy(w0_c, w0o.at[pl.ds(base, WCHUNK)])
            pltpu.sync_copy(w1_c, w1o.at[pl.ds(base, WCHUNK)])
            return carry

        lax.fori_loop(0, nch, cbody, 0)


def _sc_weights(srcb, dstb, cnts, alphat):
    mesh = plsc.VectorSubcoreMesh(core_axis_name="c", subcore_axis_name="s")
    f = pl.kernel(
        _sc_w_body,
        mesh=mesh,
        compiler_params=pltpu.CompilerParams(needs_layout_passes=False),
        out_type=[
            jax.ShapeDtypeStruct((SEGTOT,), jnp.float32),
            jax.ShapeDtypeStruct((SEGTOT,), jnp.float32),
        ],
        scratch_types=[
            pltpu.VMEM((NPAD,), jnp.float32),
            pltpu.VMEM((NPAD,), jnp.float32),
            pltpu.VMEM((NPAD,), jnp.float32),
            pltpu.VMEM((NPAD,), jnp.float32),
            pltpu.VMEM((16,), jnp.int32),
            pltpu.VMEM((WCHUNK,), jnp.int32),
            pltpu.VMEM((WCHUNK,), jnp.int32),
            pltpu.VMEM((WCHUNK,), jnp.float32),
            pltpu.VMEM((WCHUNK,), jnp.float32),
        ],
    )
    return f(srcb, dstb, cnts, alphat)


def _sc_agg_body(C, R, n_passes,
                 h, srcb, dstb, w0r, w1r, cnt_o, num0, den0, numf, denf,
                 cnts_v, src_v, dst_v, w0c, w1c,
                 csrc, cdl, cw0, cw1, rows_v, rows2, gidx, gidx2,
                 accf, accd, sem_a, sem_b):
    HC = HEADS * C
    cid = lax.axis_index("c")
    sid = lax.axis_index("s")
    wid = sid * 2 + cid
    zc16 = jnp.zeros((16,), jnp.int32)
    zf16 = jnp.zeros((16,), jnp.float32)
    iota = _iota16()

    pltpu.sync_copy(cnt_o, cnts_v)

    for p in range(n_passes):
        lo = (p * 32 + wid) * R
        b = lo // BROWS
        pltpu.sync_copy(num0.at[pl.ds(lo, R)], accf)
        pltpu.sync_copy(den0.at[pl.ds(pl.multiple_of(lo * 16, 16), R * 16)],
                        accd)

        def wbody(w, wcarry):
            sb = w * NBKT + b
            cnt = jnp.max(plsc.load_gather(
                cnts_v, [jnp.full((16,), sb, jnp.int32)]))
            seg = sb * SEGCAP
            nch = (cnt + (ACHUNK - 1)) // ACHUNK

            def cbody(c, carry0):
                cb = c * ACHUNK
                base = pl.multiple_of(seg + cb, 8)
                pltpu.sync_copy(srcb.at[pl.ds(base, ACHUNK)], src_v)
                pltpu.sync_copy(dstb.at[pl.ds(base, ACHUNK)], dst_v)
                pltpu.sync_copy(w0r.at[pl.ds(base, ACHUNK)], w0c)
                pltpu.sync_copy(w1r.at[pl.ds(base, ACHUNK)], w1c)
                rem = cnt - cb
                nv = jnp.minimum(ACHUNK // 16, (rem + 15) // 16)

                def vbody(i, off):
                    o = pl.multiple_of(i * 16, 16)
                    s16 = src_v[pl.ds(o, 16)]
                    d16 = dst_v[pl.ds(o, 16)]
                    w0 = w0c[pl.ds(o, 16)]
                    w1 = w1c[pl.ds(o, 16)]
                    dl = d16 - lo
                    m = ((dl >= 0) & (dl < R)) & (cb + o + iota < cnt)
                    mi = m.astype(jnp.int32)
                    pos = off + plsc.cumsum(mi) - mi
                    plsc.store_scatter(csrc, [pos], s16, mask=m)
                    plsc.store_scatter(cdl, [pos], dl, mask=m)
                    plsc.store_scatter(cw0, [pos], w0, mask=m)
                    plsc.store_scatter(cw1, [pos], w1, mask=m)
                    return off + jnp.sum(mi)

                off = lax.fori_loop(0, nv, vbody, jnp.int32(0))

                nsub = (off + (SUB - 1)) // SUB
                pad_end = nsub * SUB
                for k in range(SUB // 16):
                    idx = off + k * 16 + iota
                    pm = idx < pad_end
                    plsc.store_scatter(csrc, [idx], zc16, mask=pm)
                    plsc.store_scatter(cdl, [idx], zc16, mask=pm)
                    plsc.store_scatter(cw0, [idx], zf16, mask=pm)
                    plsc.store_scatter(cw1, [idx], zf16, mask=pm)

                def start_gather(j, gx, rv, sem):
                    for k in range(SUB // 16):
                        ok = pl.multiple_of(j * SUB + k * 16, 16)
                        gx[pl.ds(k * 16, 16)] = csrc[pl.ds(ok, 16)]
                    pltpu.async_copy(h.at[gx], rv, sem)

                def srow_block(j, rv):
                    def srow(r, c2):
                        i = j * SUB + r
                        si = jnp.full((16,), i, jnp.int32)
                        dlv = plsc.load_gather(cdl, [si])
                        f0 = plsc.load_gather(cw0, [si])
                        f1 = plsc.load_gather(cw1, [si])
                        wrow = jnp.where(iota == 0, f0,
                                         jnp.where(iota == 1, f1, zf16))
                        plsc.addupdate_scatter(accd, [dlv * 16 + iota], wrow)
                        for q in range(HC // 16):
                            cq = q * 16 + iota
                            f = f0 if q < (C // 16) else f1
                            v = rv[r, pl.ds(q * 16, 16)] * f
                            plsc.addupdate_scatter(accf, [dlv, cq], v)
                        return c2

                    lax.fori_loop(0, SUB, srow, 0)

                @pl.when(nsub > 0)
                def _():
                    start_gather(0, gidx, rows_v, sem_a)

                def dpair(j2, carry):
                    ja = 2 * j2

                    @pl.when(ja + 1 < nsub)
                    def _():
                        start_gather(ja + 1, gidx2, rows2, sem_b)

                    pltpu.make_async_copy(h.at[gidx], rows_v, sem_a).wait()
                    srow_block(ja, rows_v)

                    @pl.when(ja + 2 < nsub)
                    def _():
                        start_gather(ja + 2, gidx, rows_v, sem_a)

                    @pl.when(ja + 1 < nsub)
                    def _():
                        pltpu.make_async_copy(h.at[gidx2], rows2,
                                              sem_b).wait()
                        srow_block(ja + 1, rows2)

                    return carry

                lax.fori_loop(0, (nsub + 1) // 2, dpair, 0)
                return carry0

            lax.fori_loop(0, nch, cbody, 0)
            return wcarry

        lax.fori_loop(0, 16, wbody, 0)

        pltpu.sync_copy(accf, numf.at[pl.ds(lo, R)])
        pltpu.sync_copy(accd,
                        denf.at[pl.ds(pl.multiple_of(lo * 16, 16), R * 16)])


def _sc_agg(C, R, n_passes, h, srcb, dstb, w0, w1, cnts, num0, den0):
    HC = HEADS * C
    mesh = plsc.VectorSubcoreMesh(core_axis_name="c", subcore_axis_name="s")
    f = pl.kernel(
        functools.partial(_sc_agg_body, C, R, n_passes),
        mesh=mesh,
        compiler_params=pltpu.CompilerParams(needs_layout_passes=False),
        out_type=[
            jax.ShapeDtypeStruct((NPAD, HC), jnp.float32),
            jax.ShapeDtypeStruct((NPAD * 16,), jnp.float32),
        ],
        scratch_types=[
            pltpu.VMEM((16 * NBKT,), jnp.int32),
            pltpu.VMEM((ACHUNK,), jnp.int32),
            pltpu.VMEM((ACHUNK,), jnp.int32),
            pltpu.VMEM((ACHUNK,), jnp.float32),
            pltpu.VMEM((ACHUNK,), jnp.float32),
            pltpu.VMEM((ACCAP,), jnp.int32),
            pltpu.VMEM((ACCAP,), jnp.int32),
            pltpu.VMEM((ACCAP,), jnp.float32),
            pltpu.VMEM((ACCAP,), jnp.float32),
            pltpu.VMEM((SUB, HC), jnp.float32),
            pltpu.VMEM((SUB, HC), jnp.float32),
            pltpu.VMEM((SUB,), jnp.int32),
            pltpu.VMEM((SUB,), jnp.int32),
            pltpu.VMEM((R, HC), jnp.float32),
            pltpu.VMEM((R * 16,), jnp.float32),
            pltpu.SemaphoreType.DMA,
            pltpu.SemaphoreType.DMA,
        ],
    )
    return f(h, srcb, dstb, w0, w1, cnts, num0, den0)


def _stage_c_body(nbh, num_ref, den_ref, b_ref, g_ref, be_ref, o_ref):
    j = pl.program_id(0)
    num = num_ref[...]
    den = den_ref[...]
    d = jnp.where(j < nbh, den[:, 0:1], den[:, 1:2])
    pre = num / (d + 1e-16) + b_ref[...]
    mu = jnp.mean(pre, axis=0, keepdims=True)
    var = jnp.mean((pre - mu) ** 2, axis=0, keepdims=True)
    y = (pre - mu) * lax.rsqrt(var + 1e-5) * g_ref[...] + be_ref[...]
    o_ref[...] = jnp.maximum(y, 0.0)


def _stage_c(numf, denf, b, g, be):
    HC = b.shape[0]
    C = HC // HEADS
    nblk = HC // 128
    return pl.pallas_call(
        functools.partial(_stage_c_body, C // 128),
        grid=(nblk,),
        in_specs=[
            pl.BlockSpec((N, 128), lambda j: (0, j)),
            pl.BlockSpec((N, 16), lambda j: (0, 0)),
            pl.BlockSpec((1, 128), lambda j: (0, j)),
            pl.BlockSpec((1, 128), lambda j: (0, j)),
            pl.BlockSpec((1, 128), lambda j: (0, j)),
        ],
        out_specs=pl.BlockSpec((N, 128), lambda j: (0, j)),
        out_shape=jax.ShapeDtypeStruct((N, HC), jnp.float32),
    )(numf, denf, b.reshape(1, HC), g.reshape(1, HC), be.reshape(1, HC))


def kernel(x, edge_index, W1, a_src1, a_dst1, b1, g1, be1,
           W2, a_src2, a_dst2, b2, g2, be2):
    src = edge_index[0].astype(jnp.int32)
    dst = edge_index[1].astype(jnp.int32)
    srcb, dstb, cnts = _sc_bin(src, dst)

    def layer(x_in, W, a_src, a_dst, b, g, be, R, n_passes):
        x_p = jnp.pad(x_in, ((0, NPAD - N), (0, 0)))
        h, at, n0, d0 = _stage_a(x_p, W, a_src, a_dst)
        w0, w1 = _sc_weights(srcb, dstb, cnts, at)
        numf, denf = _sc_agg(a_src.shape[1], R, n_passes,
                             h, srcb, dstb, w0, w1, cnts, n0,
                             d0.reshape(-1))
        return _stage_c(numf[:N], denf.reshape(NPAD, 16)[:N], b, g, be)

    t = layer(x, W1, a_src1, a_dst1, b1, g1, be1, 160, 2)
    return layer(t, W2, a_src2, a_dst2, b2, g2, be2, 80, 4)


# speedup vs baseline: 13.4277x; 1.0386x over previous
import functools

import jax
import jax.numpy as jnp
from jax import lax
from jax.experimental import pallas as pl
from jax.experimental.pallas import tpu as pltpu
from jax.experimental.pallas import tpu_sc as plsc

HEADS = 2
N = 10000
E = 160000
NPAD = 10240
ROWBLK = 1280
EPT = E // 16
NBKT = 32
BROWS = NPAD // NBKT
MULT, SHR = 6554, 21
SEGCAP = 10240
SEGTOT = 16 * NBKT * SEGCAP
WCHUNK = 2000
ACHUNK = 512
ACCAP = ACHUNK + 16
SUB = 16


def _stage_a_body(C, x_ref, w_ref, as_ref, ad_ref, h_ref, at_ref, n0_ref,
                  d0_ref):
    x = x_ref[...]
    h = jnp.dot(x, w_ref[...], preferred_element_type=jnp.float32)
    h0 = h[:, :C]
    h1 = h[:, C:]
    a_s = as_ref[...]
    a_d = ad_ref[...]
    asrc0 = jnp.sum(h0 * a_s[0:1, :], axis=1)
    asrc1 = jnp.sum(h1 * a_s[1:2, :], axis=1)
    adst0 = jnp.sum(h0 * a_d[0:1, :], axis=1)
    adst1 = jnp.sum(h1 * a_d[1:2, :], axis=1)
    s0 = asrc0 + adst0
    s1 = asrc1 + adst1
    ws0 = jnp.exp(jnp.where(s0 >= 0, s0, 0.2 * s0))
    ws1 = jnp.exp(jnp.where(s1 >= 0, s1, 0.2 * s1))
    h_ref[...] = h
    z = jnp.zeros((1, x.shape[0]), jnp.float32)
    at_ref[...] = jnp.concatenate(
        [asrc0[None], asrc1[None], adst0[None], adst1[None], z, z, z, z], 0)
    n0_ref[...] = jnp.concatenate([h0 * ws0[:, None], h1 * ws1[:, None]], 1)
    d0_ref[...] = jnp.concatenate(
        [ws0[:, None], ws1[:, None],
         jnp.zeros((x.shape[0], 14), jnp.float32)], 1)


def _stage_a(x_p, W, a_src, a_dst):
    cin = x_p.shape[1]
    C = a_src.shape[1]
    HC = HEADS * C
    nblk = NPAD // ROWBLK
    return pl.pallas_call(
        functools.partial(_stage_a_body, C),
        grid=(nblk,),
        in_specs=[
            pl.BlockSpec((ROWBLK, cin), lambda j: (j, 0)),
            pl.BlockSpec((cin, HC), lambda j: (0, 0)),
            pl.BlockSpec((HEADS, C), lambda j: (0, 0)),
            pl.BlockSpec((HEADS, C), lambda j: (0, 0)),
        ],
        out_specs=[
            pl.BlockSpec((ROWBLK, HC), lambda j: (j, 0)),
            pl.BlockSpec((8, ROWBLK), lambda j: (0, j)),
            pl.BlockSpec((ROWBLK, HC), lambda j: (j, 0)),
            pl.BlockSpec((ROWBLK, 16), lambda j: (j, 0)),
        ],
        out_shape=[
            jax.ShapeDtypeStruct((NPAD, HC), jnp.float32),
            jax.ShapeDtypeStruct((8, NPAD), jnp.float32),
            jax.ShapeDtypeStruct((NPAD, HC), jnp.float32),
            jax.ShapeDtypeStruct((NPAD, 16), jnp.float32),
        ],
    )(x_p, W, a_src, a_dst)


def _iota16():
    return lax.iota(jnp.int32, 16)


def _sc_bin_body(srcr, dstr, src_o, dst_o, cnt_o,
                 src_v, dst_v, cs, cd, cnt_v):
    cid = lax.axis_index("c")
    sid = lax.axis_index("s")
    iota = _iota16()
    ebase = pl.multiple_of(sid * EPT, 8)
    pltpu.sync_copy(srcr.at[pl.ds(ebase, EPT)], src_v)
    pltpu.sync_copy(dstr.at[pl.ds(ebase, EPT)], dst_v)
    for bl in range(16):
        b = bl + cid * 16

        def vbody(i, off):
            o = pl.multiple_of(i * 16, 16)
            s16 = src_v[pl.ds(o, 16)]
            d16 = dst_v[pl.ds(o, 16)]
            bk = lax.shift_right_logical(d16 * MULT, SHR)
            m = bk == b
            mi = m.astype(jnp.int32)
            pos = off + plsc.cumsum(mi) - mi
            plsc.store_scatter(cs, [pos], s16, mask=m)
            plsc.store_scatter(cd, [pos], d16, mask=m)
            return off + jnp.sum(mi)

        off = lax.fori_loop(0, EPT // 16, vbody, jnp.int32(0))
        plsc.store_scatter(cnt_v, [jnp.full((16,), bl, jnp.int32)],
                           jnp.full((16,), off, jnp.int32),
                           mask=iota == 0)
        seg = pl.multiple_of((sid * NBKT + b) * SEGCAP, 8)
        pltpu.sync_copy(cs, src_o.at[pl.ds(seg, SEGCAP)])
        pltpu.sync_copy(cd, dst_o.at[pl.ds(seg, SEGCAP)])
    pltpu.sync_copy(cnt_v,
                    cnt_o.at[pl.ds(pl.multiple_of(
                        sid * NBKT + cid * 16, 8), 16)])


def _sc_bin(src, dst):
    mesh = plsc.VectorSubcoreMesh(core_axis_name="c", subcore_axis_name="s")
    f = pl.kernel(
        _sc_bin_body,
        mesh=mesh,
        compiler_params=pltpu.CompilerParams(needs_layout_passes=False),
        out_type=[
            jax.ShapeDtypeStruct((SEGTOT,), jnp.int32),
            jax.ShapeDtypeStruct((SEGTOT,), jnp.int32),
            jax.ShapeDtypeStruct((16 * NBKT,), jnp.int32),
        ],
        scratch_types=[
            pltpu.VMEM((EPT,), jnp.int32),
            pltpu.VMEM((EPT,), jnp.int32),
            pltpu.VMEM((SEGCAP,), jnp.int32),
            pltpu.VMEM((SEGCAP,), jnp.int32),
            pltpu.VMEM((16,), jnp.int32),
        ],
    )
    return f(src, dst)


def _sc_w_body(srcb, dstb, cnt_o, alphat, w0o, w1o,
               as0_v, as1_v, ad0_v, ad1_v, cnt_v, src_c, dst_c, w0_c, w1_c):
    cid = lax.axis_index("c")
    sid = lax.axis_index("s")
    pltpu.sync_copy(alphat.at[0], as0_v)
    pltpu.sync_copy(alphat.at[1], as1_v)
    pltpu.sync_copy(alphat.at[2], ad0_v)
    pltpu.sync_copy(alphat.at[3], ad1_v)
    pltpu.sync_copy(cnt_o.at[pl.ds(pl.multiple_of(
        sid * NBKT + cid * 16, 8), 16)], cnt_v)
    for bl in range(16):
        b = bl + cid * 16
        cnt = jnp.max(plsc.load_gather(
            cnt_v, [jnp.full((16,), bl, jnp.int32)]))
        seg = (sid * NBKT + b) * SEGCAP
        nch = (cnt + (WCHUNK - 1)) // WCHUNK

        def cbody(c, carry):
            base = pl.multiple_of(seg + c * WCHUNK, 8)
            pltpu.sync_copy(srcb.at[pl.ds(base, WCHUNK)], src_c)
            pltpu.sync_copy(dstb.at[pl.ds(base, WCHUNK)], dst_c)

            def vb(i, carry2):
                o = pl.multiple_of(i * 16, 16)
                s16 = jnp.clip(src_c[pl.ds(o, 16)], 0, NPAD - 1)
                d16 = jnp.clip(dst_c[pl.ds(o, 16)], 0, NPAD - 1)
                a0 = plsc.load_gather(as0_v, [s16])
                a1 = plsc.load_gather(as1_v, [s16])
                b0 = plsc.load_gather(ad0_v, [d16])
                b1 = plsc.load_gather(ad1_v, [d16])
                e0 = a0 + b0
                e1 = a1 + b1
                w0_c[pl.ds(o, 16)] = jnp.exp(
                    jnp.where(e0 >= 0, e0, 0.2 * e0))
                w1_c[pl.ds(o, 16)] = jnp.exp(
                    jnp.where(e1 >= 0, e1, 0.2 * e1))
                return carry2

            lax.fori_loop(0, WCHUNK // 16, vb, 0)
            pltpu.sync_copy(w0_c, w0o.at[pl.ds(base, WCHUNK)])
            pltpu.sync_copy(w1_c, w1o.at[pl.ds(base, WCHUNK)])
            return carry

        lax.fori_loop(0, nch, cbody, 0)


def _sc_weights(srcb, dstb, cnts, alphat):
    mesh = plsc.VectorSubcoreMesh(core_axis_name="c", subcore_axis_name="s")
    f = pl.kernel(
        _sc_w_body,
        mesh=mesh,
        compiler_params=pltpu.CompilerParams(needs_layout_passes=False),
        out_type=[
            jax.ShapeDtypeStruct((SEGTOT,), jnp.float32),
            jax.ShapeDtypeStruct((SEGTOT,), jnp.float32),
        ],
        scratch_types=[
            pltpu.VMEM((NPAD,), jnp.float32),
            pltpu.VMEM((NPAD,), jnp.float32),
            pltpu.VMEM((NPAD,), jnp.float32),
            pltpu.VMEM((NPAD,), jnp.float32),
            pltpu.VMEM((16,), jnp.int32),
            pltpu.VMEM((WCHUNK,), jnp.int32),
            pltpu.VMEM((WCHUNK,), jnp.int32),
            pltpu.VMEM((WCHUNK,), jnp.float32),
            pltpu.VMEM((WCHUNK,), jnp.float32),
        ],
    )
    return f(srcb, dstb, cnts, alphat)


def _sc_agg_body(C, R, n_passes,
                 h, srcb, dstb, w0r, w1r, cnt_o, num0, den0, numf, denf,
                 cnts_v, src_v, dst_v, w0c, w1c,
                 csrc, cdl, cw0, cw1, rows_v, rows2, gidx, gidx2,
                 accf, accd, sem_a, sem_b, sem_c):
    HC = HEADS * C
    cid = lax.axis_index("c")
    sid = lax.axis_index("s")
    wid = sid * 2 + cid
    zc16 = jnp.zeros((16,), jnp.int32)
    zf16 = jnp.zeros((16,), jnp.float32)
    iota = _iota16()

    pltpu.sync_copy(cnt_o, cnts_v)

    for p in range(n_passes):
        lo = (p * 32 + wid) * R
        b = lo // BROWS
        pltpu.sync_copy(num0.at[pl.ds(lo, R)], accf)
        pltpu.sync_copy(den0.at[pl.ds(pl.multiple_of(lo * 16, 16), R * 16)],
                        accd)

        def wbody(w, wcarry):
            sb = w * NBKT + b
            cnt = jnp.max(plsc.load_gather(
                cnts_v, [jnp.full((16,), sb, jnp.int32)]))
            seg = sb * SEGCAP
            nch = (cnt + (ACHUNK - 1)) // ACHUNK

            def cbody(c, carry0):
                cb = c * ACHUNK
                base = pl.multiple_of(seg + cb, 8)
                pltpu.async_copy(srcb.at[pl.ds(base, ACHUNK)], src_v, sem_c)
                pltpu.async_copy(dstb.at[pl.ds(base, ACHUNK)], dst_v, sem_c)
                pltpu.async_copy(w0r.at[pl.ds(base, ACHUNK)], w0c, sem_c)
                pltpu.async_copy(w1r.at[pl.ds(base, ACHUNK)], w1c, sem_c)
                for vr in (src_v, dst_v, w0c, w1c):
                    pltpu.make_async_copy(
                        srcb.at[pl.ds(base, ACHUNK)], vr, sem_c).wait()
                rem = cnt - cb
                nv = jnp.minimum(ACHUNK // 16, (rem + 15) // 16)

                def vbody(i, off):
                    o = pl.multiple_of(i * 16, 16)
                    s16 = src_v[pl.ds(o, 16)]
                    d16 = dst_v[pl.ds(o, 16)]
                    w0 = w0c[pl.ds(o, 16)]
                    w1 = w1c[pl.ds(o, 16)]
                    dl = d16 - lo
                    m = ((dl >= 0) & (dl < R)) & (cb + o + iota < cnt)
                    mi = m.astype(jnp.int32)
                    pos = off + plsc.cumsum(mi) - mi
                    plsc.store_scatter(csrc, [pos], s16, mask=m)
                    plsc.store_scatter(cdl, [pos], dl, mask=m)
                    plsc.store_scatter(cw0, [pos], w0, mask=m)
                    plsc.store_scatter(cw1, [pos], w1, mask=m)
                    return off + jnp.sum(mi)

                off = lax.fori_loop(0, nv, vbody, jnp.int32(0))

                nsub = (off + (SUB - 1)) // SUB
                pad_end = nsub * SUB
                for k in range(SUB // 16):
                    idx = off + k * 16 + iota
                    pm = idx < pad_end
                    plsc.store_scatter(csrc, [idx], zc16, mask=pm)
                    plsc.store_scatter(cdl, [idx], zc16, mask=pm)
                    plsc.store_scatter(cw0, [idx], zf16, mask=pm)
                    plsc.store_scatter(cw1, [idx], zf16, mask=pm)

                def start_gather(j, gx, rv, sem):
                    for k in range(SUB // 16):
                        ok = pl.multiple_of(j * SUB + k * 16, 16)
                        gx[pl.ds(k * 16, 16)] = csrc[pl.ds(ok, 16)]
                    pltpu.async_copy(h.at[gx], rv, sem)

                def srow_block(j, rv):
                    def srow(r, c2):
                        i = j * SUB + r
                        si = jnp.full((16,), i, jnp.int32)
                        dlv = plsc.load_gather(cdl, [si])
                        f0 = plsc.load_gather(cw0, [si])
                        f1 = plsc.load_gather(cw1, [si])
                        wrow = jnp.where(iota == 0, f0,
                                         jnp.where(iota == 1, f1, zf16))
                        plsc.addupdate_scatter(accd, [dlv * 16 + iota], wrow)
                        for q in range(HC // 16):
                            cq = q * 16 + iota
                            f = f0 if q < (C // 16) else f1
                            v = rv[r, pl.ds(q * 16, 16)] * f
                            plsc.addupdate_scatter(accf, [dlv, cq], v)
                        return c2

                    lax.fori_loop(0, SUB, srow, 0)

                @pl.when(nsub > 0)
                def _():
                    start_gather(0, gidx, rows_v, sem_a)

                def dpair(j2, carry):
                    ja = 2 * j2

                    @pl.when(ja + 1 < nsub)
                    def _():
                        start_gather(ja + 1, gidx2, rows2, sem_b)

                    pltpu.make_async_copy(h.at[gidx], rows_v, sem_a).wait()
                    srow_block(ja, rows_v)

                    @pl.when(ja + 2 < nsub)
                    def _():
                        start_gather(ja + 2, gidx, rows_v, sem_a)

                    @pl.when(ja + 1 < nsub)
                    def _():
                        pltpu.make_async_copy(h.at[gidx2], rows2,
                                              sem_b).wait()
                        srow_block(ja + 1, rows2)

                    return carry

                lax.fori_loop(0, (nsub + 1) // 2, dpair, 0)
                return carry0

            lax.fori_loop(0, nch, cbody, 0)
            return wcarry

        lax.fori_loop(0, 16, wbody, 0)

        pltpu.sync_copy(accf, numf.at[pl.ds(lo, R)])
        pltpu.sync_copy(accd,
                        denf.at[pl.ds(pl.multiple_of(lo * 16, 16), R * 16)])


def _sc_agg(C, R, n_passes, h, srcb, dstb, w0, w1, cnts, num0, den0):
    HC = HEADS * C
    mesh = plsc.VectorSubcoreMesh(core_axis_name="c", subcore_axis_name="s")
    f = pl.kernel(
        functools.partial(_sc_agg_body, C, R, n_passes),
        mesh=mesh,
        compiler_params=pltpu.CompilerParams(needs_layout_passes=False),
        out_type=[
            jax.ShapeDtypeStruct((NPAD, HC), jnp.float32),
            jax.ShapeDtypeStruct((NPAD * 16,), jnp.float32),
        ],
        scratch_types=[
            pltpu.VMEM((16 * NBKT,), jnp.int32),
            pltpu.VMEM((ACHUNK,), jnp.int32),
            pltpu.VMEM((ACHUNK,), jnp.int32),
            pltpu.VMEM((ACHUNK,), jnp.float32),
            pltpu.VMEM((ACHUNK,), jnp.float32),
            pltpu.VMEM((ACCAP,), jnp.int32),
            pltpu.VMEM((ACCAP,), jnp.int32),
            pltpu.VMEM((ACCAP,), jnp.float32),
            pltpu.VMEM((ACCAP,), jnp.float32),
            pltpu.VMEM((SUB, HC), jnp.float32),
            pltpu.VMEM((SUB, HC), jnp.float32),
            pltpu.VMEM((SUB,), jnp.int32),
            pltpu.VMEM((SUB,), jnp.int32),
            pltpu.VMEM((R, HC), jnp.float32),
            pltpu.VMEM((R * 16,), jnp.float32),
            pltpu.SemaphoreType.DMA,
            pltpu.SemaphoreType.DMA,
            pltpu.SemaphoreType.DMA,
        ],
    )
    return f(h, srcb, dstb, w0, w1, cnts, num0, den0)


def _stage_c_body(nbh, num_ref, den_ref, b_ref, g_ref, be_ref, o_ref):
    j = pl.program_id(0)
    num = num_ref[...]
    den = den_ref[...]
    d = jnp.where(j < nbh, den[:, 0:1], den[:, 1:2])
    pre = num / (d + 1e-16) + b_ref[...]
    mu = jnp.mean(pre, axis=0, keepdims=True)
    var = jnp.mean((pre - mu) ** 2, axis=0, keepdims=True)
    y = (pre - mu) * lax.rsqrt(var + 1e-5) * g_ref[...] + be_ref[...]
    o_ref[...] = jnp.maximum(y, 0.0)


def _stage_c(numf, denf, b, g, be):
    HC = b.shape[0]
    C = HC // HEADS
    nblk = HC // 128
    return pl.pallas_call(
        functools.partial(_stage_c_body, C // 128),
        grid=(nblk,),
        in_specs=[
            pl.BlockSpec((N, 128), lambda j: (0, j)),
            pl.BlockSpec((N, 16), lambda j: (0, 0)),
            pl.BlockSpec((1, 128), lambda j: (0, j)),
            pl.BlockSpec((1, 128), lambda j: (0, j)),
            pl.BlockSpec((1, 128), lambda j: (0, j)),
        ],
        out_specs=pl.BlockSpec((N, 128), lambda j: (0, j)),
        out_shape=jax.ShapeDtypeStruct((N, HC), jnp.float32),
    )(numf, denf, b.reshape(1, HC), g.reshape(1, HC), be.reshape(1, HC))


def kernel(x, edge_index, W1, a_src1, a_dst1, b1, g1, be1,
           W2, a_src2, a_dst2, b2, g2, be2):
    src = edge_index[0].astype(jnp.int32)
    dst = edge_index[1].astype(jnp.int32)
    srcb, dstb, cnts = _sc_bin(src, dst)

    def layer(x_in, W, a_src, a_dst, b, g, be, R, n_passes):
        x_p = jnp.pad(x_in, ((0, NPAD - N), (0, 0)))
        h, at, n0, d0 = _stage_a(x_p, W, a_src, a_dst)
        w0, w1 = _sc_weights(srcb, dstb, cnts, at)
        numf, denf = _sc_agg(a_src.shape[1], R, n_passes,
                             h, srcb, dstb, w0, w1, cnts, n0,
                             d0.reshape(-1))
        return _stage_c(numf[:N], denf.reshape(NPAD, 16)[:N], b, g, be)

    t = layer(x, W1, a_src1, a_dst1, b1, g1, be1, 160, 2)
    return layer(t, W2, a_src2, a_dst2, b2, g2, be2, 80, 4)
